# speedup vs baseline: 1.0787x; 1.0787x over previous
import functools

import jax
import jax.numpy as jnp
import numpy as np
from jax import lax
from jax.experimental import pallas as pl
from jax.experimental.pallas import tpu as pltpu

N = 10000
E = 160000
D = 128
HEADS = 10
OUT = 128
NEG_SLOPE = 0.2
TW = 16

ROW_BLK = 1000


def _mm1_body(x_ref, w_ref, am_ref, h_ref, al_ref):
    h = jnp.dot(x_ref[...], w_ref[...], preferred_element_type=jnp.float32)
    h_ref[...] = h
    al_ref[...] = jnp.dot(h, am_ref[...], preferred_element_type=jnp.float32)


def _mm1(x, W, a_mat, d_in, d_out):
    grid = N // ROW_BLK
    return pl.pallas_call(
        _mm1_body,
        grid=(grid,),
        in_specs=[
            pl.BlockSpec((ROW_BLK, d_in), lambda i: (i, 0)),
            pl.BlockSpec((d_in, d_out), lambda i: (0, 0)),
            pl.BlockSpec((d_out, 2 * TW), lambda i: (0, 0)),
        ],
        out_specs=[
            pl.BlockSpec((ROW_BLK, d_out), lambda i: (i, 0)),
            pl.BlockSpec((ROW_BLK, 2 * TW), lambda i: (i, 0)),
        ],
        out_shape=[
            jax.ShapeDtypeStruct((N, d_out), jnp.float32),
            jax.ShapeDtypeStruct((N, 2 * TW), jnp.float32),
        ],
    )(x, W, a_mat)


def _mid_body(agg_ref, b1_ref, w2_ref, am_ref, g_ref, al_ref):
    h1 = jnp.maximum(agg_ref[...] + b1_ref[...], 0.0)
    g = jnp.dot(h1, w2_ref[...], preferred_element_type=jnp.float32)
    g_ref[...] = g
    al_ref[...] = jnp.dot(g, am_ref[...], preferred_element_type=jnp.float32)


def _mid(agg1, b1, W2, a_mat2):
    grid = N // ROW_BLK
    return pl.pallas_call(
        _mid_body,
        grid=(grid,),
        in_specs=[
            pl.BlockSpec((ROW_BLK, HEADS * D), lambda i: (i, 0)),
            pl.BlockSpec((1, HEADS * D), lambda i: (0, 0)),
            pl.BlockSpec((HEADS * D, OUT), lambda i: (0, 0)),
            pl.BlockSpec((OUT, 2 * TW), lambda i: (0, 0)),
        ],
        out_specs=[
            pl.BlockSpec((ROW_BLK, OUT), lambda i: (i, 0)),
            pl.BlockSpec((ROW_BLK, 2 * TW), lambda i: (i, 0)),
        ],
        out_shape=[
            jax.ShapeDtypeStruct((N, OUT), jnp.float32),
            jax.ShapeDtypeStruct((N, 2 * TW), jnp.float32),
        ],
    )(agg1, b1.reshape(1, HEADS * D), W2, a_mat2)


def _fin_body(agg_ref, b2_ref, fcw_ref, fcb_ref, y_ref, gr_ref, mx_ref):
    i = pl.program_id(0)
    h2 = jnp.maximum(agg_ref[...] + b2_ref[...], 0.0)
    y_ref[...] = (
        jnp.dot(h2, fcw_ref[...], preferred_element_type=jnp.float32)
        + fcb_ref[...]
    )
    blockmax = jnp.max(h2, axis=0, keepdims=True)

    @pl.when(i == 0)
    def _():
        mx_ref[...] = blockmax

    @pl.when(i > 0)
    def _():
        mx_ref[...] = jnp.maximum(mx_ref[...], blockmax)

    @pl.when(i == pl.num_programs(0) - 1)
    def _():
        gr_ref[...] = (
            jnp.dot(mx_ref[...], fcw_ref[...], preferred_element_type=jnp.float32)
            + fcb_ref[...]
        )


def _fin(agg2, b2, fcW, fcb):
    grid = N // ROW_BLK
    return pl.pallas_call(
        _fin_body,
        grid=(grid,),
        in_specs=[
            pl.BlockSpec((ROW_BLK, OUT), lambda i: (i, 0)),
            pl.BlockSpec((1, OUT), lambda i: (0, 0)),
            pl.BlockSpec((OUT, OUT), lambda i: (0, 0)),
            pl.BlockSpec((1, OUT), lambda i: (0, 0)),
        ],
        out_specs=[
            pl.BlockSpec((ROW_BLK, OUT), lambda i: (i, 0)),
            pl.BlockSpec((1, OUT), lambda i: (0, 0)),
        ],
        out_shape=[
            jax.ShapeDtypeStruct((N, OUT), jnp.float32),
            jax.ShapeDtypeStruct((1, OUT), jnp.float32),
        ],
        scratch_shapes=[pltpu.VMEM((1, OUT), jnp.float32)],
    )(agg2, b2.reshape(1, OUT), fcW, fcb.reshape(1, OUT))


def _pack_alpha(a_src, a_dst, d_out, heads, head_dim):
    m = jnp.zeros((d_out, 2 * TW), jnp.float32)
    for h in range(heads):
        m = m.at[h * head_dim:(h + 1) * head_dim, h].set(a_src[h])
        m = m.at[h * head_dim:(h + 1) * head_dim, TW + h].set(a_dst[h])
    return m


def _edge_stage(al, h, src, dst, heads, head_dim):
    es = al[:, :heads]
    ed = al[:, TW:TW + heads]
    e = es[src] + ed[dst]
    e = jnp.where(e >= 0, e, NEG_SLOPE * e)
    ee = jnp.exp(e)
    denom = jax.ops.segment_sum(ee, dst, num_segments=N)
    att = ee / denom[dst]
    msg = h.reshape(N, heads, head_dim)[src] * att[:, :, None]
    agg = jax.ops.segment_sum(msg, dst, num_segments=N)
    return agg.reshape(N, heads * head_dim)


def kernel(x, edge_index, graph_id, W1, a_src1, a_dst1, b1, W2, a_src2,
           a_dst2, b2, fcW, fcb):
    loops = jnp.arange(N, dtype=edge_index.dtype)
    src = jnp.concatenate([edge_index[0], loops])
    dst = jnp.concatenate([edge_index[1], loops])

    a_mat1 = _pack_alpha(a_src1, a_dst1, HEADS * D, HEADS, D)
    a_mat2 = _pack_alpha(a_src2, a_dst2, OUT, 1, OUT)

    h1mm, al1 = _mm1(x, W1, a_mat1, D, HEADS * D)
    agg1 = _edge_stage(al1, h1mm, src, dst, HEADS, D)
    g2, al2 = _mid(agg1, b1, W2, a_mat2)
    agg2 = _edge_stage(al2, g2, src, dst, 1, OUT)
    y, gr = _fin(agg2, b2, fcW, fcb)
    return y[None, :, :], gr


# speedup vs baseline: 6.7356x; 6.2444x over previous
import functools

import jax
import jax.numpy as jnp
import numpy as np
from jax import lax
from jax.experimental import pallas as pl
from jax.experimental.pallas import tpu as pltpu
from jax.experimental.pallas import tpu_sc as plsc

N = 10000
E = 160000
D = 128
HEADS = 10
OUT = 128
NEG_SLOPE = 0.2
TW = 16

ROW_BLK = 1000

NSC = 2
NTILES = 16
EP = 172032
C_TILE = EP // (NSC * NTILES)
CB = 128
NPAD = 10240
STRIPE_A = NPAD // NTILES
AW = 128

_sc_mesh = plsc.VectorSubcoreMesh(core_axis_name="c", subcore_axis_name="s")


def _denom_body(src_hbm, dst_hbm, al_hbm, out_hbm,
                sidx, didx, esb, edb, acc, sem1, sem2):
    c = lax.axis_index("c")
    s = lax.axis_index("s")
    wid = s * NSC + c

    def zrow(i, _):
        def zcol(j, _):
            edb[i, pl.ds(j * 16, 16)] = jnp.zeros((16,), jnp.float32)
            return 0
        return lax.fori_loop(0, AW // 16, zcol, 0)
    lax.fori_loop(0, CB, zrow, 0)
    for z in range(STRIPE_A // CB):
        pltpu.sync_copy(edb, acc.at[pl.ds(s * STRIPE_A + z * CB, CB)])
    plsc.subcore_barrier()

    def batch(b, _):
        off = wid * C_TILE + b * CB
        pltpu.sync_copy(src_hbm.at[pl.ds(off, CB)], sidx)
        pltpu.sync_copy(dst_hbm.at[pl.ds(off, CB)], didx)
        cp1 = pltpu.async_copy(al_hbm.at[sidx], esb, sem1)
        cp2 = pltpu.async_copy(al_hbm.at[didx], edb, sem2)
        cp1.wait()
        cp2.wait()

        def row(j, _):
            e = esb[j, pl.ds(0, 16)] + edb[j, pl.ds(16, 16)]
            e = jnp.where(e >= 0.0, e, NEG_SLOPE * e)
            edb[j, pl.ds(0, 16)] = jnp.exp(e)
            return 0
        lax.fori_loop(0, CB, row, 0, unroll=4)
        pltpu.sync_copy(edb, acc.at[didx], add=True)
        return 0
    lax.fori_loop(0, C_TILE // CB, batch, 0)
    plsc.subcore_barrier()

    pltpu.sync_copy(acc.at[pl.ds(s * STRIPE_A, STRIPE_A)],
                    out_hbm.at[c, pl.ds(s * STRIPE_A, STRIPE_A)])


_denom_kernel = functools.partial(
    pl.kernel,
    out_type=jax.ShapeDtypeStruct((NSC, NPAD, AW), jnp.float32),
    mesh=_sc_mesh,
    scratch_types=[
        pltpu.VMEM((CB,), jnp.int32),
        pltpu.VMEM((CB,), jnp.int32),
        pltpu.VMEM((CB, AW), jnp.float32),
        pltpu.VMEM((CB, AW), jnp.float32),
        pltpu.VMEM_SHARED((NPAD, AW), jnp.float32),
        pltpu.SemaphoreType.DMA,
        pltpu.SemaphoreType.DMA,
    ],
)(_denom_body)


def _att_body(src_hbm, dst_hbm, al_hbm, dd_hbm, att_hbm,
              sidx, didx, esb, ddb, atb, sem1, sem2):
    c = lax.axis_index("c")
    s = lax.axis_index("s")
    wid = s * NSC + c

    def batch(b, _):
        off = wid * C_TILE + b * CB
        pltpu.sync_copy(src_hbm.at[pl.ds(off, CB)], sidx)
        pltpu.sync_copy(dst_hbm.at[pl.ds(off, CB)], didx)
        cp1 = pltpu.async_copy(al_hbm.at[sidx], esb, sem1)
        cp2 = pltpu.async_copy(dd_hbm.at[didx], ddb, sem2)
        cp1.wait()
        cp2.wait()

        def row(j, _):
            e = esb[j, pl.ds(0, 16)] + ddb[j, pl.ds(16, 16)]
            e = jnp.where(e >= 0.0, e, NEG_SLOPE * e)
            atb[pl.ds(j * 16, 16)] = jnp.exp(e) / ddb[j, pl.ds(0, 16)]
            return 0
        lax.fori_loop(0, CB, row, 0, unroll=4)
        pltpu.sync_copy(atb, att_hbm.at[pl.ds(off * 16, CB * 16)])
        return 0
    lax.fori_loop(0, C_TILE // CB, batch, 0)


_att_kernel = functools.partial(
    pl.kernel,
    out_type=jax.ShapeDtypeStruct((EP * 16,), jnp.float32),
    mesh=_sc_mesh,
    scratch_types=[
        pltpu.VMEM((CB,), jnp.int32),
        pltpu.VMEM((CB,), jnp.int32),
        pltpu.VMEM((CB, AW), jnp.float32),
        pltpu.VMEM((CB, AW), jnp.float32),
        pltpu.VMEM((CB * 16,), jnp.float32),
        pltpu.SemaphoreType.DMA,
        pltpu.SemaphoreType.DMA,
    ],
)(_att_body)


def _scale_rows(hbuf, atb, lane):
    def srow(r, _):
        av = atb[pl.ds(r * 16, 16)]
        a = av[lane]
        for v in range(AW // 16):
            hbuf[r, pl.ds(v * 16, 16)] = hbuf[r, pl.ds(v * 16, 16)] * a
        return 0
    lax.fori_loop(0, CB, srow, 0, unroll=2)


def _agg1_body(src_hbm, dst_hbm, att_hbm, hsp_hbm, out_hbm,
               sidx, didx, sgi, atb, hbuf, acc, sem1):
    c = lax.axis_index("c")
    s = lax.axis_index("s")
    estripe = EP // NTILES

    for k in range(HEADS // NSC):
        fb = c * (HEADS // NSC) + k

        def zrow(r, _):
            for v in range(AW // 16):
                hbuf[r, pl.ds(v * 16, 16)] = jnp.zeros((16,), jnp.float32)
            return 0
        lax.fori_loop(0, CB, zrow, 0)
        for z in range(STRIPE_A // CB):
            pltpu.sync_copy(hbuf, acc.at[pl.ds(s * STRIPE_A + z * CB, CB)])
        plsc.subcore_barrier()

        def chunk(q, _):
            off = s * estripe + q * CB
            pltpu.sync_copy(src_hbm.at[pl.ds(off, CB)], sidx)
            pltpu.sync_copy(dst_hbm.at[pl.ds(off, CB)], didx)
            pltpu.sync_copy(att_hbm.at[pl.ds(off * 16, CB * 16)], atb)

            def addoff(j, _):
                sgi[pl.ds(j * 16, 16)] = (
                    sidx[pl.ds(j * 16, 16)] + fb * N)
                return 0
            lax.fori_loop(0, CB // 16, addoff, 0)
            pltpu.async_copy(hsp_hbm.at[sgi], hbuf, sem1).wait()

            @pl.when(c == 0)
            def _():
                _scale_rows(hbuf, atb, k)

            @pl.when(c == 1)
            def _():
                _scale_rows(hbuf, atb, HEADS // NSC + k)

            pltpu.sync_copy(hbuf, acc.at[didx], add=True)
            return 0
        lax.fori_loop(0, estripe // CB, chunk, 0)
        plsc.subcore_barrier()

        pltpu.sync_copy(acc.at[pl.ds(s * STRIPE_A, STRIPE_A)],
                        out_hbm.at[fb, pl.ds(s * STRIPE_A, STRIPE_A)])
        plsc.subcore_barrier()


_agg1_kernel = functools.partial(
    pl.kernel,
    out_type=jax.ShapeDtypeStruct((HEADS, NPAD, AW), jnp.float32),
    mesh=_sc_mesh,
    scratch_types=[
        pltpu.VMEM((CB,), jnp.int32),
        pltpu.VMEM((CB,), jnp.int32),
        pltpu.VMEM((CB,), jnp.int32),
        pltpu.VMEM((CB * 16,), jnp.float32),
        pltpu.VMEM((CB, AW), jnp.float32),
        pltpu.VMEM_SHARED((NPAD, AW), jnp.float32),
        pltpu.SemaphoreType.DMA,
    ],
)(_agg1_body)


def _agg2_body(src_hbm, dst_hbm, att_hbm, h_hbm, out_hbm,
               sidx, didx, atb, hbuf, acc, sem1):
    c = lax.axis_index("c")
    s = lax.axis_index("s")
    wid = s * NSC + c

    def zrow(r, _):
        for v in range(AW // 16):
            hbuf[r, pl.ds(v * 16, 16)] = jnp.zeros((16,), jnp.float32)
        return 0
    lax.fori_loop(0, CB, zrow, 0)
    for z in range(STRIPE_A // CB):
        pltpu.sync_copy(hbuf, acc.at[pl.ds(s * STRIPE_A + z * CB, CB)])
    plsc.subcore_barrier()

    def chunk(q, _):
        off = wid * C_TILE + q * CB
        pltpu.sync_copy(src_hbm.at[pl.ds(off, CB)], sidx)
        pltpu.sync_copy(dst_hbm.at[pl.ds(off, CB)], didx)
        pltpu.sync_copy(att_hbm.at[pl.ds(off * 16, CB * 16)], atb)
        pltpu.async_copy(h_hbm.at[sidx], hbuf, sem1).wait()
        _scale_rows(hbuf, atb, 0)
        pltpu.sync_copy(hbuf, acc.at[didx], add=True)
        return 0
    lax.fori_loop(0, C_TILE // CB, chunk, 0)
    plsc.subcore_barrier()

    pltpu.sync_copy(acc.at[pl.ds(s * STRIPE_A, STRIPE_A)],
                    out_hbm.at[c, pl.ds(s * STRIPE_A, STRIPE_A)])


_agg2_kernel = functools.partial(
    pl.kernel,
    out_type=jax.ShapeDtypeStruct((NSC, NPAD, AW), jnp.float32),
    mesh=_sc_mesh,
    scratch_types=[
        pltpu.VMEM((CB,), jnp.int32),
        pltpu.VMEM((CB,), jnp.int32),
        pltpu.VMEM((CB * 16,), jnp.float32),
        pltpu.VMEM((CB, AW), jnp.float32),
        pltpu.VMEM_SHARED((NPAD, AW), jnp.float32),
        pltpu.SemaphoreType.DMA,
    ],
)(_agg2_body)


_COMB_BLK = 1024


def _comb_body(al_ref, d0_ref, d1_ref, dd_ref):
    colmask = lax.broadcasted_iota(jnp.int32, (_COMB_BLK, AW), 1) < TW
    dd_ref[...] = jnp.where(colmask, d0_ref[...] + d1_ref[...], al_ref[...])


def _combine(alp, d0, d1):
    return pl.pallas_call(
        _comb_body,
        grid=(NPAD // _COMB_BLK,),
        in_specs=[pl.BlockSpec((_COMB_BLK, AW), lambda i: (i, 0))] * 3,
        out_specs=pl.BlockSpec((_COMB_BLK, AW), lambda i: (i, 0)),
        out_shape=jax.ShapeDtypeStruct((NPAD, AW), jnp.float32),
    )(alp, d0, d1)


def _mm1_body(x_ref, w_ref, am_ref, h_ref, al_ref):
    h = jnp.dot(x_ref[...], w_ref[...], preferred_element_type=jnp.float32)
    h_ref[...] = h
    al_ref[...] = jnp.dot(h, am_ref[...], preferred_element_type=jnp.float32)


def _mm1(x, W, a_mat, d_in, d_out):
    grid = N // ROW_BLK
    return pl.pallas_call(
        _mm1_body,
        grid=(grid,),
        in_specs=[
            pl.BlockSpec((ROW_BLK, d_in), lambda i: (i, 0)),
            pl.BlockSpec((d_in, d_out), lambda i: (0, 0)),
            pl.BlockSpec((d_out, AW), lambda i: (0, 0)),
        ],
        out_specs=[
            pl.BlockSpec((ROW_BLK, d_out), lambda i: (i, 0)),
            pl.BlockSpec((ROW_BLK, AW), lambda i: (i, 0)),
        ],
        out_shape=[
            jax.ShapeDtypeStruct((N, d_out), jnp.float32),
            jax.ShapeDtypeStruct((N, AW), jnp.float32),
        ],
    )(x, W, a_mat)


def _mid_body(agg_ref, b1_ref, w2_ref, am_ref, g_ref, al_ref):
    h1 = jnp.maximum(agg_ref[...] + b1_ref[...], 0.0)
    g = jnp.dot(h1, w2_ref[...], preferred_element_type=jnp.float32)
    g_ref[...] = g
    al_ref[...] = jnp.dot(g, am_ref[...], preferred_element_type=jnp.float32)


def _mid(agg1, b1, W2, a_mat2):
    grid = N // ROW_BLK
    return pl.pallas_call(
        _mid_body,
        grid=(grid,),
        in_specs=[
            pl.BlockSpec((ROW_BLK, HEADS * D), lambda i: (i, 0)),
            pl.BlockSpec((1, HEADS * D), lambda i: (0, 0)),
            pl.BlockSpec((HEADS * D, OUT), lambda i: (0, 0)),
            pl.BlockSpec((OUT, AW), lambda i: (0, 0)),
        ],
        out_specs=[
            pl.BlockSpec((ROW_BLK, OUT), lambda i: (i, 0)),
            pl.BlockSpec((ROW_BLK, AW), lambda i: (i, 0)),
        ],
        out_shape=[
            jax.ShapeDtypeStruct((N, OUT), jnp.float32),
            jax.ShapeDtypeStruct((N, AW), jnp.float32),
        ],
    )(agg1, b1.reshape(1, HEADS * D), W2, a_mat2)


def _fin_body(p0_ref, p1_ref, b2_ref, fcw_ref, fcb_ref, y_ref, gr_ref, mx_ref):
    i = pl.program_id(0)
    h2 = jnp.maximum(p0_ref[...] + p1_ref[...] + b2_ref[...], 0.0)
    y_ref[...] = (
        jnp.dot(h2, fcw_ref[...], preferred_element_type=jnp.float32)
        + fcb_ref[...]
    )
    blockmax = jnp.max(h2, axis=0, keepdims=True)

    @pl.when(i == 0)
    def _():
        mx_ref[...] = blockmax

    @pl.when(i > 0)
    def _():
        mx_ref[...] = jnp.maximum(mx_ref[...], blockmax)

    @pl.when(i == pl.num_programs(0) - 1)
    def _():
        gr_ref[...] = (
            jnp.dot(mx_ref[...], fcw_ref[...], preferred_element_type=jnp.float32)
            + fcb_ref[...]
        )


def _fin(p0, p1, b2, fcW, fcb):
    grid = N // ROW_BLK
    return pl.pallas_call(
        _fin_body,
        grid=(grid,),
        in_specs=[
            pl.BlockSpec((ROW_BLK, OUT), lambda i: (i, 0)),
            pl.BlockSpec((ROW_BLK, OUT), lambda i: (i, 0)),
            pl.BlockSpec((1, OUT), lambda i: (0, 0)),
            pl.BlockSpec((OUT, OUT), lambda i: (0, 0)),
            pl.BlockSpec((1, OUT), lambda i: (0, 0)),
        ],
        out_specs=[
            pl.BlockSpec((ROW_BLK, OUT), lambda i: (i, 0)),
            pl.BlockSpec((1, OUT), lambda i: (0, 0)),
        ],
        out_shape=[
            jax.ShapeDtypeStruct((N, OUT), jnp.float32),
            jax.ShapeDtypeStruct((1, OUT), jnp.float32),
        ],
        scratch_shapes=[pltpu.VMEM((1, OUT), jnp.float32)],
    )(p0, p1, b2.reshape(1, OUT), fcW, fcb.reshape(1, OUT))


def _pack_alpha(a_src, a_dst, d_out, heads, head_dim):
    m = jnp.zeros((d_out, AW), jnp.float32)
    for h in range(heads):
        m = m.at[h * head_dim:(h + 1) * head_dim, h].set(a_src[h])
        m = m.at[h * head_dim:(h + 1) * head_dim, TW + h].set(a_dst[h])
    return m


def kernel(x, edge_index, graph_id, W1, a_src1, a_dst1, b1, W2, a_src2,
           a_dst2, b2, fcW, fcb):
    loops = jnp.arange(N, dtype=edge_index.dtype)
    pad = jnp.zeros((EP - E - N,), jnp.int32)
    src = jnp.concatenate([edge_index[0], loops])
    dst = jnp.concatenate([edge_index[1], loops])
    src_p = jnp.concatenate([src, pad])
    dst_p = jnp.concatenate([dst, pad + N])

    a_mat1 = _pack_alpha(a_src1, a_dst1, HEADS * D, HEADS, D)
    a_mat2 = _pack_alpha(a_src2, a_dst2, OUT, 1, OUT)

    def tables(al):
        return jnp.pad(al, ((0, NPAD - N), (0, 0)))

    h1mm, al1 = _mm1(x, W1, a_mat1, D, HEADS * D)
    al1p = tables(al1)
    dns1 = _denom_kernel(src_p, dst_p, al1p)
    dd1 = _combine(al1p, dns1[0], dns1[1])
    att1 = _att_kernel(src_p, dst_p, al1p, dd1)
    hsp = h1mm.reshape(N, HEADS, D).swapaxes(0, 1).reshape(HEADS * N, D)
    out1 = _agg1_kernel(src_p, dst_p, att1, hsp)
    agg1 = out1[:, :N, :].swapaxes(0, 1).reshape(N, HEADS * D)
    g2, al2 = _mid(agg1, b1, W2, a_mat2)
    al2p = tables(al2)
    dns2 = _denom_kernel(src_p, dst_p, al2p)
    dd2 = _combine(al2p, dns2[0], dns2[1])
    att2 = _att_kernel(src_p, dst_p, al2p, dd2)
    agg2p = _agg2_kernel(src_p, dst_p, att2, g2)
    y, gr = _fin(agg2p[0, :N], agg2p[1, :N], b2, fcW, fcb)
    return y[None, :, :], gr


# speedup vs baseline: 7.9205x; 1.1759x over previous
import functools

import jax
import jax.numpy as jnp
import numpy as np
from jax import lax
from jax.experimental import pallas as pl
from jax.experimental.pallas import tpu as pltpu
from jax.experimental.pallas import tpu_sc as plsc

N = 10000
E = 160000
D = 128
HEADS = 10
OUT = 128
NEG_SLOPE = 0.2
TW = 16

ROW_BLK = 1000

NSC = 2
NTILES = 16
EP = 172032
C_TILE = EP // (NSC * NTILES)
CB = 128
NPAD = 10240
STRIPE_A = NPAD // NTILES
AW = 128

_sc_mesh = plsc.VectorSubcoreMesh(core_axis_name="c", subcore_axis_name="s")


def _denom_body(src_hbm, dst_hbm, al_hbm, out_hbm,
                sidx, didx, esb, edb, acc, sem1, sem2):
    c = lax.axis_index("c")
    s = lax.axis_index("s")
    wid = s * NSC + c

    def zrow(i, _):
        def zcol(j, _):
            edb[i, pl.ds(j * 16, 16)] = jnp.zeros((16,), jnp.float32)
            return 0
        return lax.fori_loop(0, AW // 16, zcol, 0)
    lax.fori_loop(0, CB, zrow, 0)
    for z in range(STRIPE_A // CB):
        pltpu.sync_copy(edb, acc.at[pl.ds(s * STRIPE_A + z * CB, CB)])
    plsc.subcore_barrier()

    def batch(b, _):
        off = wid * C_TILE + b * CB
        pltpu.sync_copy(src_hbm.at[pl.ds(off, CB)], sidx)
        pltpu.sync_copy(dst_hbm.at[pl.ds(off, CB)], didx)
        cp1 = pltpu.async_copy(al_hbm.at[sidx], esb, sem1)
        cp2 = pltpu.async_copy(al_hbm.at[didx], edb, sem2)
        cp1.wait()
        cp2.wait()

        def row(j, _):
            e = esb[j, pl.ds(0, 16)] + edb[j, pl.ds(16, 16)]
            e = jnp.where(e >= 0.0, e, NEG_SLOPE * e)
            edb[j, pl.ds(0, 16)] = jnp.exp(e)
            return 0
        lax.fori_loop(0, CB, row, 0, unroll=4)
        pltpu.sync_copy(edb, acc.at[didx], add=True)
        return 0
    lax.fori_loop(0, C_TILE // CB, batch, 0)
    plsc.subcore_barrier()

    pltpu.sync_copy(acc.at[pl.ds(s * STRIPE_A, STRIPE_A)],
                    out_hbm.at[c, pl.ds(s * STRIPE_A, STRIPE_A)])


_denom_kernel = functools.partial(
    pl.kernel,
    out_type=jax.ShapeDtypeStruct((NSC, NPAD, AW), jnp.float32),
    mesh=_sc_mesh,
    scratch_types=[
        pltpu.VMEM((CB,), jnp.int32),
        pltpu.VMEM((CB,), jnp.int32),
        pltpu.VMEM((CB, AW), jnp.float32),
        pltpu.VMEM((CB, AW), jnp.float32),
        pltpu.VMEM_SHARED((NPAD, AW), jnp.float32),
        pltpu.SemaphoreType.DMA,
        pltpu.SemaphoreType.DMA,
    ],
)(_denom_body)


def _att_body(src_hbm, dst_hbm, al_hbm, dd_hbm, att_hbm,
              sidx, didx, esb, ddb, atb, sem1, sem2):
    c = lax.axis_index("c")
    s = lax.axis_index("s")
    wid = s * NSC + c

    def batch(b, _):
        off = wid * C_TILE + b * CB
        pltpu.sync_copy(src_hbm.at[pl.ds(off, CB)], sidx)
        pltpu.sync_copy(dst_hbm.at[pl.ds(off, CB)], didx)
        cp1 = pltpu.async_copy(al_hbm.at[sidx], esb, sem1)
        cp2 = pltpu.async_copy(dd_hbm.at[didx], ddb, sem2)
        cp1.wait()
        cp2.wait()

        def row(j, _):
            e = esb[j, pl.ds(0, 16)] + ddb[j, pl.ds(16, 16)]
            e = jnp.where(e >= 0.0, e, NEG_SLOPE * e)
            atb[pl.ds(j * 16, 16)] = jnp.exp(e) / ddb[j, pl.ds(0, 16)]
            return 0
        lax.fori_loop(0, CB, row, 0, unroll=4)
        pltpu.sync_copy(atb, att_hbm.at[pl.ds(off * 16, CB * 16)])
        return 0
    lax.fori_loop(0, C_TILE // CB, batch, 0)


_att_kernel = functools.partial(
    pl.kernel,
    out_type=jax.ShapeDtypeStruct((EP * 16,), jnp.float32),
    mesh=_sc_mesh,
    scratch_types=[
        pltpu.VMEM((CB,), jnp.int32),
        pltpu.VMEM((CB,), jnp.int32),
        pltpu.VMEM((CB, AW), jnp.float32),
        pltpu.VMEM((CB, AW), jnp.float32),
        pltpu.VMEM((CB * 16,), jnp.float32),
        pltpu.SemaphoreType.DMA,
        pltpu.SemaphoreType.DMA,
    ],
)(_att_body)


def _scale_rows(hbuf, atb, lane):
    def srow(r, _):
        av = atb[pl.ds(r * 16, 16)]
        a = av[lane]
        for v in range(AW // 16):
            hbuf[r, pl.ds(v * 16, 16)] = hbuf[r, pl.ds(v * 16, 16)] * a
        return 0
    lax.fori_loop(0, CB, srow, 0, unroll=4)


def _agg1_body(src_hbm, dst_hbm, att_hbm, hsp_hbm, out_hbm,
               sidx0, sidx1, didx0, didx1, sgi0, sgi1, atb0, atb1,
               hbuf0, hbuf1, acc, sem0, sem1):
    c = lax.axis_index("c")
    s = lax.axis_index("s")
    estripe = EP // NTILES
    NCH = estripe // CB
    sidx = (sidx0, sidx1)
    didx = (didx0, didx1)
    sgi = (sgi0, sgi1)
    atb = (atb0, atb1)
    hbuf = (hbuf0, hbuf1)
    sem = (sem0, sem1)

    def load_chunk(b, off):
        pltpu.sync_copy(src_hbm.at[pl.ds(off, CB)], sidx[b])
        pltpu.sync_copy(dst_hbm.at[pl.ds(off, CB)], didx[b])
        pltpu.sync_copy(att_hbm.at[pl.ds(off * 16, CB * 16)], atb[b])

    def issue_gather(b, fb):
        def addoff(j, _):
            sgi[b][pl.ds(j * 16, 16)] = sidx[b][pl.ds(j * 16, 16)] + fb * N
            return 0
        lax.fori_loop(0, CB // 16, addoff, 0)
        pltpu.async_copy(hsp_hbm.at[sgi[b]], hbuf[b], sem[b])

    def wait_gather(b):
        pltpu.make_async_copy(hsp_hbm.at[sgi[b]], hbuf[b], sem[b]).wait()

    for k in range(HEADS // NSC):
        fb = c * (HEADS // NSC) + k

        def zrow(r, _):
            for v in range(AW // 16):
                hbuf0[r, pl.ds(v * 16, 16)] = jnp.zeros((16,), jnp.float32)
            return 0
        lax.fori_loop(0, CB, zrow, 0)
        for z in range(STRIPE_A // CB):
            pltpu.sync_copy(hbuf0, acc.at[pl.ds(s * STRIPE_A + z * CB, CB)])
        plsc.subcore_barrier()

        base = s * estripe
        load_chunk(0, base)
        issue_gather(0, fb)

        def pair(q2, _):
            q = q2 * 2

            def half(b, qa):
                qn = jnp.minimum(qa + 1, NCH - 1)
                load_chunk(1 - b, base + qn * CB)
                issue_gather(1 - b, fb)
                wait_gather(b)

                @pl.when(c == 0)
                def _():
                    _scale_rows(hbuf[b], atb[b], k)

                @pl.when(c == 1)
                def _():
                    _scale_rows(hbuf[b], atb[b], HEADS // NSC + k)

                pltpu.sync_copy(hbuf[b], acc.at[didx[b]], add=True)
            half(0, q)
            half(1, q + 1)
            return 0
        lax.fori_loop(0, NCH // 2, pair, 0)
        wait_gather(0)
        plsc.subcore_barrier()

        pltpu.sync_copy(acc.at[pl.ds(s * STRIPE_A, STRIPE_A)],
                        out_hbm.at[fb, pl.ds(s * STRIPE_A, STRIPE_A)])
        plsc.subcore_barrier()


_agg1_kernel = functools.partial(
    pl.kernel,
    out_type=jax.ShapeDtypeStruct((HEADS, NPAD, AW), jnp.float32),
    mesh=_sc_mesh,
    scratch_types=[
        pltpu.VMEM((CB,), jnp.int32),
        pltpu.VMEM((CB,), jnp.int32),
        pltpu.VMEM((CB,), jnp.int32),
        pltpu.VMEM((CB,), jnp.int32),
        pltpu.VMEM((CB,), jnp.int32),
        pltpu.VMEM((CB,), jnp.int32),
        pltpu.VMEM((CB * 16,), jnp.float32),
        pltpu.VMEM((CB * 16,), jnp.float32),
        pltpu.VMEM((CB, AW), jnp.float32),
        pltpu.VMEM((CB, AW), jnp.float32),
        pltpu.VMEM_SHARED((NPAD, AW), jnp.float32),
        pltpu.SemaphoreType.DMA,
        pltpu.SemaphoreType.DMA,
    ],
)(_agg1_body)


def _agg2_body(src_hbm, dst_hbm, att_hbm, h_hbm, out_hbm,
               sidx, didx, atb, hbuf, acc, sem1):
    c = lax.axis_index("c")
    s = lax.axis_index("s")
    wid = s * NSC + c

    def zrow(r, _):
        for v in range(AW // 16):
            hbuf[r, pl.ds(v * 16, 16)] = jnp.zeros((16,), jnp.float32)
        return 0
    lax.fori_loop(0, CB, zrow, 0)
    for z in range(STRIPE_A // CB):
        pltpu.sync_copy(hbuf, acc.at[pl.ds(s * STRIPE_A + z * CB, CB)])
    plsc.subcore_barrier()

    def chunk(q, _):
        off = wid * C_TILE + q * CB
        pltpu.sync_copy(src_hbm.at[pl.ds(off, CB)], sidx)
        pltpu.sync_copy(dst_hbm.at[pl.ds(off, CB)], didx)
        pltpu.sync_copy(att_hbm.at[pl.ds(off * 16, CB * 16)], atb)
        pltpu.async_copy(h_hbm.at[sidx], hbuf, sem1).wait()
        _scale_rows(hbuf, atb, 0)
        pltpu.sync_copy(hbuf, acc.at[didx], add=True)
        return 0
    lax.fori_loop(0, C_TILE // CB, chunk, 0)
    plsc.subcore_barrier()

    pltpu.sync_copy(acc.at[pl.ds(s * STRIPE_A, STRIPE_A)],
                    out_hbm.at[c, pl.ds(s * STRIPE_A, STRIPE_A)])


_agg2_kernel = functools.partial(
    pl.kernel,
    out_type=jax.ShapeDtypeStruct((NSC, NPAD, AW), jnp.float32),
    mesh=_sc_mesh,
    scratch_types=[
        pltpu.VMEM((CB,), jnp.int32),
        pltpu.VMEM((CB,), jnp.int32),
        pltpu.VMEM((CB * 16,), jnp.float32),
        pltpu.VMEM((CB, AW), jnp.float32),
        pltpu.VMEM_SHARED((NPAD, AW), jnp.float32),
        pltpu.SemaphoreType.DMA,
    ],
)(_agg2_body)


_COMB_BLK = 1024


def _comb_body(al_ref, d0_ref, d1_ref, dd_ref):
    colmask = lax.broadcasted_iota(jnp.int32, (_COMB_BLK, AW), 1) < TW
    dd_ref[...] = jnp.where(colmask, d0_ref[...] + d1_ref[...], al_ref[...])


def _combine(alp, d0, d1):
    return pl.pallas_call(
        _comb_body,
        grid=(NPAD // _COMB_BLK,),
        in_specs=[pl.BlockSpec((_COMB_BLK, AW), lambda i: (i, 0))] * 3,
        out_specs=pl.BlockSpec((_COMB_BLK, AW), lambda i: (i, 0)),
        out_shape=jax.ShapeDtypeStruct((NPAD, AW), jnp.float32),
    )(alp, d0, d1)


def _mm1_body(x_ref, w_ref, am_ref, h_ref, al_ref):
    h = jnp.dot(x_ref[...], w_ref[...], preferred_element_type=jnp.float32)
    h_ref[...] = h
    al_ref[...] = jnp.dot(h, am_ref[...], preferred_element_type=jnp.float32)


def _mm1(x, W, a_mat, d_in, d_out):
    grid = N // ROW_BLK
    return pl.pallas_call(
        _mm1_body,
        grid=(grid,),
        in_specs=[
            pl.BlockSpec((ROW_BLK, d_in), lambda i: (i, 0)),
            pl.BlockSpec((d_in, d_out), lambda i: (0, 0)),
            pl.BlockSpec((d_out, AW), lambda i: (0, 0)),
        ],
        out_specs=[
            pl.BlockSpec((ROW_BLK, d_out), lambda i: (i, 0)),
            pl.BlockSpec((ROW_BLK, AW), lambda i: (i, 0)),
        ],
        out_shape=[
            jax.ShapeDtypeStruct((N, d_out), jnp.float32),
            jax.ShapeDtypeStruct((N, AW), jnp.float32),
        ],
    )(x, W, a_mat)


def _mid_body(agg_ref, b1_ref, w2_ref, am_ref, g_ref, al_ref):
    h1 = jnp.maximum(agg_ref[...] + b1_ref[...], 0.0)
    g = jnp.dot(h1, w2_ref[...], preferred_element_type=jnp.float32)
    g_ref[...] = g
    al_ref[...] = jnp.dot(g, am_ref[...], preferred_element_type=jnp.float32)


def _mid(agg1, b1, W2, a_mat2):
    grid = N // ROW_BLK
    return pl.pallas_call(
        _mid_body,
        grid=(grid,),
        in_specs=[
            pl.BlockSpec((ROW_BLK, HEADS * D), lambda i: (i, 0)),
            pl.BlockSpec((1, HEADS * D), lambda i: (0, 0)),
            pl.BlockSpec((HEADS * D, OUT), lambda i: (0, 0)),
            pl.BlockSpec((OUT, AW), lambda i: (0, 0)),
        ],
        out_specs=[
            pl.BlockSpec((ROW_BLK, OUT), lambda i: (i, 0)),
            pl.BlockSpec((ROW_BLK, AW), lambda i: (i, 0)),
        ],
        out_shape=[
            jax.ShapeDtypeStruct((N, OUT), jnp.float32),
            jax.ShapeDtypeStruct((N, AW), jnp.float32),
        ],
    )(agg1, b1.reshape(1, HEADS * D), W2, a_mat2)


def _fin_body(p0_ref, p1_ref, b2_ref, fcw_ref, fcb_ref, y_ref, gr_ref, mx_ref):
    i = pl.program_id(0)
    h2 = jnp.maximum(p0_ref[...] + p1_ref[...] + b2_ref[...], 0.0)
    y_ref[...] = (
        jnp.dot(h2, fcw_ref[...], preferred_element_type=jnp.float32)
        + fcb_ref[...]
    )
    blockmax = jnp.max(h2, axis=0, keepdims=True)

    @pl.when(i == 0)
    def _():
        mx_ref[...] = blockmax

    @pl.when(i > 0)
    def _():
        mx_ref[...] = jnp.maximum(mx_ref[...], blockmax)

    @pl.when(i == pl.num_programs(0) - 1)
    def _():
        gr_ref[...] = (
            jnp.dot(mx_ref[...], fcw_ref[...], preferred_element_type=jnp.float32)
            + fcb_ref[...]
        )


def _fin(p0, p1, b2, fcW, fcb):
    grid = N // ROW_BLK
    return pl.pallas_call(
        _fin_body,
        grid=(grid,),
        in_specs=[
            pl.BlockSpec((ROW_BLK, OUT), lambda i: (i, 0)),
            pl.BlockSpec((ROW_BLK, OUT), lambda i: (i, 0)),
            pl.BlockSpec((1, OUT), lambda i: (0, 0)),
            pl.BlockSpec((OUT, OUT), lambda i: (0, 0)),
            pl.BlockSpec((1, OUT), lambda i: (0, 0)),
        ],
        out_specs=[
            pl.BlockSpec((ROW_BLK, OUT), lambda i: (i, 0)),
            pl.BlockSpec((1, OUT), lambda i: (0, 0)),
        ],
        out_shape=[
            jax.ShapeDtypeStruct((N, OUT), jnp.float32),
            jax.ShapeDtypeStruct((1, OUT), jnp.float32),
        ],
        scratch_shapes=[pltpu.VMEM((1, OUT), jnp.float32)],
    )(p0, p1, b2.reshape(1, OUT), fcW, fcb.reshape(1, OUT))


def _pack_alpha(a_src, a_dst, d_out, heads, head_dim):
    m = jnp.zeros((d_out, AW), jnp.float32)
    for h in range(heads):
        m = m.at[h * head_dim:(h + 1) * head_dim, h].set(a_src[h])
        m = m.at[h * head_dim:(h + 1) * head_dim, TW + h].set(a_dst[h])
    return m


def kernel(x, edge_index, graph_id, W1, a_src1, a_dst1, b1, W2, a_src2,
           a_dst2, b2, fcW, fcb):
    loops = jnp.arange(N, dtype=edge_index.dtype)
    pad = jnp.zeros((EP - E - N,), jnp.int32)
    src = jnp.concatenate([edge_index[0], loops])
    dst = jnp.concatenate([edge_index[1], loops])
    src_p = jnp.concatenate([src, pad])
    dst_p = jnp.concatenate([dst, pad + N])

    a_mat1 = _pack_alpha(a_src1, a_dst1, HEADS * D, HEADS, D)
    a_mat2 = _pack_alpha(a_src2, a_dst2, OUT, 1, OUT)

    def tables(al):
        return jnp.pad(al, ((0, NPAD - N), (0, 0)))

    h1mm, al1 = _mm1(x, W1, a_mat1, D, HEADS * D)
    al1p = tables(al1)
    dns1 = _denom_kernel(src_p, dst_p, al1p)
    dd1 = _combine(al1p, dns1[0], dns1[1])
    att1 = _att_kernel(src_p, dst_p, al1p, dd1)
    hsp = h1mm.reshape(N, HEADS, D).swapaxes(0, 1).reshape(HEADS * N, D)
    out1 = _agg1_kernel(src_p, dst_p, att1, hsp)
    agg1 = out1[:, :N, :].swapaxes(0, 1).reshape(N, HEADS * D)
    g2, al2 = _mid(agg1, b1, W2, a_mat2)
    al2p = tables(al2)
    dns2 = _denom_kernel(src_p, dst_p, al2p)
    dd2 = _combine(al2p, dns2[0], dns2[1])
    att2 = _att_kernel(src_p, dst_p, al2p, dd2)
    agg2p = _agg2_kernel(src_p, dst_p, att2, g2)
    y, gr = _fin(agg2p[0, :N], agg2p[1, :N], b2, fcW, fcb)
    return y[None, :, :], gr


# speedup vs baseline: 8.7490x; 1.1046x over previous
import functools

import jax
import jax.numpy as jnp
import numpy as np
from jax import lax
from jax.experimental import pallas as pl
from jax.experimental.pallas import tpu as pltpu
from jax.experimental.pallas import tpu_sc as plsc

N = 10000
E = 160000
D = 128
HEADS = 10
OUT = 128
NEG_SLOPE = 0.2
TW = 16

ROW_BLK = 1000

NSC = 2
NTILES = 16
EP = 172032
C_TILE = EP // (NSC * NTILES)
CB = 128
NPAD = 10240
STRIPE_A = NPAD // NTILES
AW = 128

_sc_mesh = plsc.VectorSubcoreMesh(core_axis_name="c", subcore_axis_name="s")


def _denom_body(src_hbm, dst_hbm, al_hbm, out_hbm,
                sidx, didx, esb, edb, acc, sem1, sem2):
    c = lax.axis_index("c")
    s = lax.axis_index("s")
    wid = s * NSC + c

    def zrow(i, _):
        def zcol(j, _):
            edb[i, pl.ds(j * 16, 16)] = jnp.zeros((16,), jnp.float32)
            return 0
        return lax.fori_loop(0, AW // 16, zcol, 0)
    lax.fori_loop(0, CB, zrow, 0)
    for z in range(STRIPE_A // CB):
        pltpu.sync_copy(edb, acc.at[pl.ds(s * STRIPE_A + z * CB, CB)])
    plsc.subcore_barrier()

    def batch(b, _):
        off = wid * C_TILE + b * CB
        pltpu.sync_copy(src_hbm.at[pl.ds(off, CB)], sidx)
        pltpu.sync_copy(dst_hbm.at[pl.ds(off, CB)], didx)
        cp1 = pltpu.async_copy(al_hbm.at[sidx], esb, sem1)
        cp2 = pltpu.async_copy(al_hbm.at[didx], edb, sem2)
        cp1.wait()
        cp2.wait()

        def row(j, _):
            e = esb[j, pl.ds(0, 16)] + edb[j, pl.ds(16, 16)]
            e = jnp.where(e >= 0.0, e, NEG_SLOPE * e)
            edb[j, pl.ds(0, 16)] = jnp.exp(e)
            return 0
        lax.fori_loop(0, CB, row, 0, unroll=4)
        pltpu.sync_copy(edb, acc.at[didx], add=True)
        return 0
    lax.fori_loop(0, C_TILE // CB, batch, 0)
    plsc.subcore_barrier()

    pltpu.sync_copy(acc.at[pl.ds(s * STRIPE_A, STRIPE_A)],
                    out_hbm.at[c, pl.ds(s * STRIPE_A, STRIPE_A)])


_denom_kernel = functools.partial(
    pl.kernel,
    out_type=jax.ShapeDtypeStruct((NSC, NPAD, AW), jnp.float32),
    mesh=_sc_mesh,
    scratch_types=[
        pltpu.VMEM((CB,), jnp.int32),
        pltpu.VMEM((CB,), jnp.int32),
        pltpu.VMEM((CB, AW), jnp.float32),
        pltpu.VMEM((CB, AW), jnp.float32),
        pltpu.VMEM_SHARED((NPAD, AW), jnp.float32),
        pltpu.SemaphoreType.DMA,
        pltpu.SemaphoreType.DMA,
    ],
)(_denom_body)


def _att_body(src_hbm, dst_hbm, al_hbm, dd_hbm, att_hbm,
              sidx, didx, esb, ddb, atb, sem1, sem2):
    c = lax.axis_index("c")
    s = lax.axis_index("s")
    wid = s * NSC + c

    def batch(b, _):
        off = wid * C_TILE + b * CB
        pltpu.sync_copy(src_hbm.at[pl.ds(off, CB)], sidx)
        pltpu.sync_copy(dst_hbm.at[pl.ds(off, CB)], didx)
        cp1 = pltpu.async_copy(al_hbm.at[sidx], esb, sem1)
        cp2 = pltpu.async_copy(dd_hbm.at[didx], ddb, sem2)
        cp1.wait()
        cp2.wait()

        def row(j, _):
            e = esb[j, pl.ds(0, 16)] + ddb[j, pl.ds(16, 16)]
            e = jnp.where(e >= 0.0, e, NEG_SLOPE * e)
            atb[pl.ds(j * 16, 16)] = jnp.exp(e) / ddb[j, pl.ds(0, 16)]
            return 0
        lax.fori_loop(0, CB, row, 0, unroll=4)
        pltpu.sync_copy(atb, att_hbm.at[pl.ds(off * 16, CB * 16)])
        return 0
    lax.fori_loop(0, C_TILE // CB, batch, 0)


_att_kernel = functools.partial(
    pl.kernel,
    out_type=jax.ShapeDtypeStruct((EP * 16,), jnp.float32),
    mesh=_sc_mesh,
    scratch_types=[
        pltpu.VMEM((CB,), jnp.int32),
        pltpu.VMEM((CB,), jnp.int32),
        pltpu.VMEM((CB, AW), jnp.float32),
        pltpu.VMEM((CB, AW), jnp.float32),
        pltpu.VMEM((CB * 16,), jnp.float32),
        pltpu.SemaphoreType.DMA,
        pltpu.SemaphoreType.DMA,
    ],
)(_att_body)


def _scale_rows(hbuf, atb, lane):
    def srow(r, _):
        av = atb[pl.ds(r * 16, 16)]
        a = av[lane]
        for v in range(AW // 16):
            hbuf[r, pl.ds(v * 16, 16)] = hbuf[r, pl.ds(v * 16, 16)] * a
        return 0
    lax.fori_loop(0, CB, srow, 0, unroll=4)


def _agg1_body(src_hbm, dst_hbm, att_hbm, hsp_hbm, out_hbm,
               sidx0, sidx1, didx0, didx1, sgi0, sgi1, atb0, atb1,
               hbuf0, hbuf1, acc, sem0, sem1, lsem0, lsem1, dsem0, dsem1):
    c = lax.axis_index("c")
    s = lax.axis_index("s")
    estripe = EP // NTILES
    NCH = estripe // CB
    sidx = (sidx0, sidx1)
    didx = (didx0, didx1)
    sgi = (sgi0, sgi1)
    atb = (atb0, atb1)
    hbuf = (hbuf0, hbuf1)
    sem = (sem0, sem1)
    lsem = (lsem0, lsem1)
    dsem = (dsem0, dsem1)

    def issue_loads(b, off):
        pltpu.async_copy(src_hbm.at[pl.ds(off, CB)], sidx[b], lsem[b])
        pltpu.async_copy(att_hbm.at[pl.ds(off * 16, CB * 16)], atb[b], lsem[b])

    def wait_loads(b):
        pltpu.make_async_copy(src_hbm.at[pl.ds(0, CB)], sidx[b], lsem[b]).wait()
        pltpu.make_async_copy(att_hbm.at[pl.ds(0, CB * 16)], atb[b], lsem[b]).wait()

    def issue_gather(b, fb):
        def addoff(j, _):
            sgi[b][pl.ds(j * 16, 16)] = sidx[b][pl.ds(j * 16, 16)] + fb * N
            return 0
        lax.fori_loop(0, CB // 16, addoff, 0)
        pltpu.async_copy(hsp_hbm.at[sgi[b]], hbuf[b], sem[b])

    def wait_gather(b):
        pltpu.make_async_copy(hsp_hbm.at[sgi[b]], hbuf[b], sem[b]).wait()

    for k in range(HEADS // NSC):
        fb = c * (HEADS // NSC) + k

        def zrow(r, _):
            for v in range(AW // 16):
                hbuf0[r, pl.ds(v * 16, 16)] = jnp.zeros((16,), jnp.float32)
            return 0
        lax.fori_loop(0, CB, zrow, 0)
        for z in range(STRIPE_A // CB):
            pltpu.sync_copy(hbuf0, acc.at[pl.ds(s * STRIPE_A + z * CB, CB)])
        plsc.subcore_barrier()

        base = s * estripe
        pltpu.sync_copy(dst_hbm.at[pl.ds(base, CB)], didx[0])
        issue_loads(0, base)
        wait_loads(0)
        issue_gather(0, fb)
        pltpu.sync_copy(dst_hbm.at[pl.ds(base + CB, CB)], didx[1])
        issue_loads(1, base + CB)

        def pair(q2, _):
            q = q2 * 2

            def half(b, qa):
                qn1 = jnp.minimum(qa + 1, NCH - 1)
                qn2 = jnp.minimum(qa + 2, NCH - 1)
                wait_loads(1 - b)
                issue_gather(1 - b, fb)
                wait_gather(b)

                @pl.when(c == 0)
                def _():
                    _scale_rows(hbuf[b], atb[b], k)

                @pl.when(c == 1)
                def _():
                    _scale_rows(hbuf[b], atb[b], HEADS // NSC + k)

                issue_loads(b, base + qn2 * CB)
                pltpu.sync_copy(hbuf[b], acc.at[didx[b]], add=True)
                pltpu.async_copy(dst_hbm.at[pl.ds(base + qn2 * CB, CB)],
                                 didx[b], dsem[b])
                pltpu.make_async_copy(dst_hbm.at[pl.ds(0, CB)], didx[b],
                                      dsem[b]).wait()
            half(0, q)
            half(1, q + 1)
            return 0
        lax.fori_loop(0, NCH // 2, pair, 0)
        wait_loads(1)
        wait_gather(0)
        plsc.subcore_barrier()

        pltpu.sync_copy(acc.at[pl.ds(s * STRIPE_A, STRIPE_A)],

                        out_hbm.at[fb, pl.ds(s * STRIPE_A, STRIPE_A)])
        plsc.subcore_barrier()


_agg1_kernel = functools.partial(
    pl.kernel,
    out_type=jax.ShapeDtypeStruct((HEADS, NPAD, AW), jnp.float32),
    mesh=_sc_mesh,
    scratch_types=[
        pltpu.VMEM((CB,), jnp.int32),
        pltpu.VMEM((CB,), jnp.int32),
        pltpu.VMEM((CB,), jnp.int32),
        pltpu.VMEM((CB,), jnp.int32),
        pltpu.VMEM((CB,), jnp.int32),
        pltpu.VMEM((CB,), jnp.int32),
        pltpu.VMEM((CB * 16,), jnp.float32),
        pltpu.VMEM((CB * 16,), jnp.float32),
        pltpu.VMEM((CB, AW), jnp.float32),
        pltpu.VMEM((CB, AW), jnp.float32),
        pltpu.VMEM_SHARED((NPAD, AW), jnp.float32),
        pltpu.SemaphoreType.DMA,
        pltpu.SemaphoreType.DMA,
        pltpu.SemaphoreType.DMA,
        pltpu.SemaphoreType.DMA,
        pltpu.SemaphoreType.DMA,
        pltpu.SemaphoreType.DMA,
    ],
)(_agg1_body)


def _agg2_body(src_hbm, dst_hbm, att_hbm, h_hbm, out_hbm,
               sidx, didx, atb, hbuf, acc, sem1):
    c = lax.axis_index("c")
    s = lax.axis_index("s")
    wid = s * NSC + c

    def zrow(r, _):
        for v in range(AW // 16):
            hbuf[r, pl.ds(v * 16, 16)] = jnp.zeros((16,), jnp.float32)
        return 0
    lax.fori_loop(0, CB, zrow, 0)
    for z in range(STRIPE_A // CB):
        pltpu.sync_copy(hbuf, acc.at[pl.ds(s * STRIPE_A + z * CB, CB)])
    plsc.subcore_barrier()

    def chunk(q, _):
        off = wid * C_TILE + q * CB
        pltpu.sync_copy(src_hbm.at[pl.ds(off, CB)], sidx)
        pltpu.sync_copy(dst_hbm.at[pl.ds(off, CB)], didx)
        pltpu.sync_copy(att_hbm.at[pl.ds(off * 16, CB * 16)], atb)
        pltpu.async_copy(h_hbm.at[sidx], hbuf, sem1).wait()
        _scale_rows(hbuf, atb, 0)
        pltpu.sync_copy(hbuf, acc.at[didx], add=True)
        return 0
    lax.fori_loop(0, C_TILE // CB, chunk, 0)
    plsc.subcore_barrier()

    pltpu.sync_copy(acc.at[pl.ds(s * STRIPE_A, STRIPE_A)],
                    out_hbm.at[c, pl.ds(s * STRIPE_A, STRIPE_A)])


_agg2_kernel = functools.partial(
    pl.kernel,
    out_type=jax.ShapeDtypeStruct((NSC, NPAD, AW), jnp.float32),
    mesh=_sc_mesh,
    scratch_types=[
        pltpu.VMEM((CB,), jnp.int32),
        pltpu.VMEM((CB,), jnp.int32),
        pltpu.VMEM((CB * 16,), jnp.float32),
        pltpu.VMEM((CB, AW), jnp.float32),
        pltpu.VMEM_SHARED((NPAD, AW), jnp.float32),
        pltpu.SemaphoreType.DMA,
    ],
)(_agg2_body)


_COMB_BLK = 1024


def _comb_body(al_ref, d0_ref, d1_ref, dd_ref):
    colmask = lax.broadcasted_iota(jnp.int32, (_COMB_BLK, AW), 1) < TW
    dd_ref[...] = jnp.where(colmask, d0_ref[...] + d1_ref[...], al_ref[...])


def _combine(alp, d0, d1):
    return pl.pallas_call(
        _comb_body,
        grid=(NPAD // _COMB_BLK,),
        in_specs=[pl.BlockSpec((_COMB_BLK, AW), lambda i: (i, 0))] * 3,
        out_specs=pl.BlockSpec((_COMB_BLK, AW), lambda i: (i, 0)),
        out_shape=jax.ShapeDtypeStruct((NPAD, AW), jnp.float32),
    )(alp, d0, d1)


def _mm1_body(x_ref, w_ref, am_ref, h_ref, al_ref):
    h = jnp.dot(x_ref[...], w_ref[...], preferred_element_type=jnp.float32)
    h_ref[...] = h
    al_ref[...] = jnp.dot(h, am_ref[...], preferred_element_type=jnp.float32)


def _mm1(x, W, a_mat, d_in, d_out):
    grid = N // ROW_BLK
    return pl.pallas_call(
        _mm1_body,
        grid=(grid,),
        in_specs=[
            pl.BlockSpec((ROW_BLK, d_in), lambda i: (i, 0)),
            pl.BlockSpec((d_in, d_out), lambda i: (0, 0)),
            pl.BlockSpec((d_out, AW), lambda i: (0, 0)),
        ],
        out_specs=[
            pl.BlockSpec((ROW_BLK, d_out), lambda i: (i, 0)),
            pl.BlockSpec((ROW_BLK, AW), lambda i: (i, 0)),
        ],
        out_shape=[
            jax.ShapeDtypeStruct((N, d_out), jnp.float32),
            jax.ShapeDtypeStruct((N, AW), jnp.float32),
        ],
    )(x, W, a_mat)


def _mid_body(agg_ref, b1_ref, w2_ref, am_ref, g_ref, al_ref):
    h1 = jnp.maximum(agg_ref[...] + b1_ref[...], 0.0)
    g = jnp.dot(h1, w2_ref[...], preferred_element_type=jnp.float32)
    g_ref[...] = g
    al_ref[...] = jnp.dot(g, am_ref[...], preferred_element_type=jnp.float32)


def _mid(agg1, b1, W2, a_mat2):
    grid = N // ROW_BLK
    return pl.pallas_call(
        _mid_body,
        grid=(grid,),
        in_specs=[
            pl.BlockSpec((ROW_BLK, HEADS * D), lambda i: (i, 0)),
            pl.BlockSpec((1, HEADS * D), lambda i: (0, 0)),
            pl.BlockSpec((HEADS * D, OUT), lambda i: (0, 0)),
            pl.BlockSpec((OUT, AW), lambda i: (0, 0)),
        ],
        out_specs=[
            pl.BlockSpec((ROW_BLK, OUT), lambda i: (i, 0)),
            pl.BlockSpec((ROW_BLK, AW), lambda i: (i, 0)),
        ],
        out_shape=[
            jax.ShapeDtypeStruct((N, OUT), jnp.float32),
            jax.ShapeDtypeStruct((N, AW), jnp.float32),
        ],
    )(agg1, b1.reshape(1, HEADS * D), W2, a_mat2)


def _fin_body(p0_ref, p1_ref, b2_ref, fcw_ref, fcb_ref, y_ref, gr_ref, mx_ref):
    i = pl.program_id(0)
    h2 = jnp.maximum(p0_ref[...] + p1_ref[...] + b2_ref[...], 0.0)
    y_ref[...] = (
        jnp.dot(h2, fcw_ref[...], preferred_element_type=jnp.float32)
        + fcb_ref[...]
    )
    blockmax = jnp.max(h2, axis=0, keepdims=True)

    @pl.when(i == 0)
    def _():
        mx_ref[...] = blockmax

    @pl.when(i > 0)
    def _():
        mx_ref[...] = jnp.maximum(mx_ref[...], blockmax)

    @pl.when(i == pl.num_programs(0) - 1)
    def _():
        gr_ref[...] = (
            jnp.dot(mx_ref[...], fcw_ref[...], preferred_element_type=jnp.float32)
            + fcb_ref[...]
        )


def _fin(p0, p1, b2, fcW, fcb):
    grid = N // ROW_BLK
    return pl.pallas_call(
        _fin_body,
        grid=(grid,),
        in_specs=[
            pl.BlockSpec((ROW_BLK, OUT), lambda i: (i, 0)),
            pl.BlockSpec((ROW_BLK, OUT), lambda i: (i, 0)),
            pl.BlockSpec((1, OUT), lambda i: (0, 0)),
            pl.BlockSpec((OUT, OUT), lambda i: (0, 0)),
            pl.BlockSpec((1, OUT), lambda i: (0, 0)),
        ],
        out_specs=[
            pl.BlockSpec((ROW_BLK, OUT), lambda i: (i, 0)),
            pl.BlockSpec((1, OUT), lambda i: (0, 0)),
        ],
        out_shape=[
            jax.ShapeDtypeStruct((N, OUT), jnp.float32),
            jax.ShapeDtypeStruct((1, OUT), jnp.float32),
        ],
        scratch_shapes=[pltpu.VMEM((1, OUT), jnp.float32)],
    )(p0, p1, b2.reshape(1, OUT), fcW, fcb.reshape(1, OUT))


def _pack_alpha(a_src, a_dst, d_out, heads, head_dim):
    m = jnp.zeros((d_out, AW), jnp.float32)
    for h in range(heads):
        m = m.at[h * head_dim:(h + 1) * head_dim, h].set(a_src[h])
        m = m.at[h * head_dim:(h + 1) * head_dim, TW + h].set(a_dst[h])
    return m


def kernel(x, edge_index, graph_id, W1, a_src1, a_dst1, b1, W2, a_src2,
           a_dst2, b2, fcW, fcb):
    loops = jnp.arange(N, dtype=edge_index.dtype)
    pad = jnp.zeros((EP - E - N,), jnp.int32)
    src = jnp.concatenate([edge_index[0], loops])
    dst = jnp.concatenate([edge_index[1], loops])
    src_p = jnp.concatenate([src, pad])
    dst_p = jnp.concatenate([dst, pad + N])

    a_mat1 = _pack_alpha(a_src1, a_dst1, HEADS * D, HEADS, D)
    a_mat2 = _pack_alpha(a_src2, a_dst2, OUT, 1, OUT)

    def tables(al):
        return jnp.pad(al, ((0, NPAD - N), (0, 0)))

    h1mm, al1 = _mm1(x, W1, a_mat1, D, HEADS * D)
    al1p = tables(al1)
    dns1 = _denom_kernel(src_p, dst_p, al1p)
    dd1 = _combine(al1p, dns1[0], dns1[1])
    att1 = _att_kernel(src_p, dst_p, al1p, dd1)
    hsp = h1mm.reshape(N, HEADS, D).swapaxes(0, 1).reshape(HEADS * N, D)
    out1 = _agg1_kernel(src_p, dst_p, att1, hsp)
    agg1 = out1[:, :N, :].swapaxes(0, 1).reshape(N, HEADS * D)
    g2, al2 = _mid(agg1, b1, W2, a_mat2)
    al2p = tables(al2)
    dns2 = _denom_kernel(src_p, dst_p, al2p)
    dd2 = _combine(al2p, dns2[0], dns2[1])
    att2 = _att_kernel(src_p, dst_p, al2p, dd2)
    agg2p = _agg2_kernel(src_p, dst_p, att2, g2)
    y, gr = _fin(agg2p[0, :N], agg2p[1, :N], b2, fcW, fcb)
    return y[None, :, :], gr


# speedup vs baseline: 9.3568x; 1.0695x over previous
import functools

import jax
import jax.numpy as jnp
import numpy as np
from jax import lax
from jax.experimental import pallas as pl
from jax.experimental.pallas import tpu as pltpu
from jax.experimental.pallas import tpu_sc as plsc

N = 10000
E = 160000
D = 128
HEADS = 10
OUT = 128
NEG_SLOPE = 0.2
TW = 16

ROW_BLK = 1000

NSC = 2
NTILES = 16
EP = 172032
C_TILE = EP // (NSC * NTILES)
CB = 128
NPAD = 10240
STRIPE_A = NPAD // NTILES
AW = 128

_sc_mesh = plsc.VectorSubcoreMesh(core_axis_name="c", subcore_axis_name="s")


def _denom_body(src_hbm, dst_hbm, al_hbm, out_hbm,
                sidx, didx, esb, edb, acc, sem1, sem2):
    c = lax.axis_index("c")
    s = lax.axis_index("s")
    wid = s * NSC + c

    def zrow(i, _):
        def zcol(j, _):
            edb[i, pl.ds(j * 16, 16)] = jnp.zeros((16,), jnp.float32)
            return 0
        return lax.fori_loop(0, AW // 16, zcol, 0)
    lax.fori_loop(0, CB, zrow, 0)
    for z in range(STRIPE_A // CB):
        pltpu.sync_copy(edb, acc.at[pl.ds(s * STRIPE_A + z * CB, CB)])
    plsc.subcore_barrier()

    def batch(b, _):
        off = wid * C_TILE + b * CB
        pltpu.sync_copy(src_hbm.at[pl.ds(off, CB)], sidx)
        pltpu.sync_copy(dst_hbm.at[pl.ds(off, CB)], didx)
        cp1 = pltpu.async_copy(al_hbm.at[sidx], esb, sem1)
        cp2 = pltpu.async_copy(al_hbm.at[didx], edb, sem2)
        cp1.wait()
        cp2.wait()

        def row(j, _):
            e = esb[j, pl.ds(0, 16)] + edb[j, pl.ds(16, 16)]
            e = jnp.where(e >= 0.0, e, NEG_SLOPE * e)
            edb[j, pl.ds(0, 16)] = jnp.exp(e)
            return 0
        lax.fori_loop(0, CB, row, 0, unroll=4)
        pltpu.sync_copy(edb, acc.at[didx], add=True)
        return 0
    lax.fori_loop(0, C_TILE // CB, batch, 0)
    plsc.subcore_barrier()

    pltpu.sync_copy(acc.at[pl.ds(s * STRIPE_A, STRIPE_A)],
                    out_hbm.at[c, pl.ds(s * STRIPE_A, STRIPE_A)])


_denom_kernel = functools.partial(
    pl.kernel,
    out_type=jax.ShapeDtypeStruct((NSC, NPAD, AW), jnp.float32),
    mesh=_sc_mesh,
    scratch_types=[
        pltpu.VMEM((CB,), jnp.int32),
        pltpu.VMEM((CB,), jnp.int32),
        pltpu.VMEM((CB, AW), jnp.float32),
        pltpu.VMEM((CB, AW), jnp.float32),
        pltpu.VMEM_SHARED((NPAD, AW), jnp.float32),
        pltpu.SemaphoreType.DMA,
        pltpu.SemaphoreType.DMA,
    ],
)(_denom_body)


def _att_body(src_hbm, dst_hbm, al_hbm, dd_hbm, att_hbm,
              sidx0, sidx1, didx0, didx1, esb0, esb1, ddb0, ddb1,
              atb0, atb1, gs0, gs1, gd0, gd1):
    c = lax.axis_index("c")
    s = lax.axis_index("s")
    wid = s * NSC + c
    NCH = C_TILE // CB
    sidx = (sidx0, sidx1)
    didx = (didx0, didx1)
    esb = (esb0, esb1)
    ddb = (ddb0, ddb1)
    atb = (atb0, atb1)
    gs = (gs0, gs1)
    gd = (gd0, gd1)

    def start_chunk(b, off):
        pltpu.sync_copy(src_hbm.at[pl.ds(off, CB)], sidx[b])
        pltpu.sync_copy(dst_hbm.at[pl.ds(off, CB)], didx[b])
        pltpu.async_copy(al_hbm.at[sidx[b]], esb[b], gs[b])
        pltpu.async_copy(dd_hbm.at[didx[b]], ddb[b], gd[b])

    def wait_chunk(b):
        pltpu.make_async_copy(al_hbm.at[sidx[b]], esb[b], gs[b]).wait()
        pltpu.make_async_copy(dd_hbm.at[didx[b]], ddb[b], gd[b]).wait()

    base = wid * C_TILE
    start_chunk(0, base)

    def pair(q2, _):
        q = q2 * 2

        def half(b, qa):
            qn = jnp.minimum(qa + 1, NCH - 1)
            start_chunk(1 - b, base + qn * CB)
            wait_chunk(b)

            def row(j, _):
                e = esb[b][j, pl.ds(0, 16)] + ddb[b][j, pl.ds(16, 16)]
                e = jnp.where(e >= 0.0, e, NEG_SLOPE * e)
                atb[b][pl.ds(j * 16, 16)] = (
                    jnp.exp(e) / ddb[b][j, pl.ds(0, 16)])
                return 0
            lax.fori_loop(0, CB, row, 0, unroll=4)
            pltpu.sync_copy(atb[b],
                            att_hbm.at[pl.ds((base + qa * CB) * 16, CB * 16)])
        half(0, q)
        half(1, q + 1)
        return 0
    lax.fori_loop(0, NCH // 2, pair, 0)
    wait_chunk(0)


_att_kernel = functools.partial(
    pl.kernel,
    out_type=jax.ShapeDtypeStruct((EP * 16,), jnp.float32),
    mesh=_sc_mesh,
    scratch_types=(
        [pltpu.VMEM((CB,), jnp.int32)] * 4
        + [pltpu.VMEM((CB, AW), jnp.float32)] * 4
        + [pltpu.VMEM((CB * 16,), jnp.float32)] * 2
        + [pltpu.SemaphoreType.DMA] * 4
    ),
)(_att_body)


def _scale_rows(hbuf, atb, lane):
    def srow(r, _):
        av = atb[pl.ds(r * 16, 16)]
        a = av[lane]
        for v in range(AW // 16):
            hbuf[r, pl.ds(v * 16, 16)] = hbuf[r, pl.ds(v * 16, 16)] * a
        return 0
    lax.fori_loop(0, CB, srow, 0, unroll=4)


def _agg1_body(src_hbm, dst_hbm, att_hbm, hsp_hbm, out_hbm,
               sidx0, sidx1, didx0, didx1, sgi0, sgi1, atb0, atb1,
               hbuf0, hbuf1, acc, sem0, sem1, lsem0, lsem1, dsem0, dsem1):
    c = lax.axis_index("c")
    s = lax.axis_index("s")
    estripe = EP // NTILES
    NCH = estripe // CB
    sidx = (sidx0, sidx1)
    didx = (didx0, didx1)
    sgi = (sgi0, sgi1)
    atb = (atb0, atb1)
    hbuf = (hbuf0, hbuf1)
    sem = (sem0, sem1)
    lsem = (lsem0, lsem1)
    dsem = (dsem0, dsem1)

    def issue_loads(b, off):
        pltpu.async_copy(src_hbm.at[pl.ds(off, CB)], sidx[b], lsem[b])
        pltpu.async_copy(att_hbm.at[pl.ds(off * 16, CB * 16)], atb[b], lsem[b])

    def wait_loads(b):
        pltpu.make_async_copy(src_hbm.at[pl.ds(0, CB)], sidx[b], lsem[b]).wait()
        pltpu.make_async_copy(att_hbm.at[pl.ds(0, CB * 16)], atb[b], lsem[b]).wait()

    def issue_gather(b, fb):
        def addoff(j, _):
            sgi[b][pl.ds(j * 16, 16)] = sidx[b][pl.ds(j * 16, 16)] + fb * N
            return 0
        lax.fori_loop(0, CB // 16, addoff, 0)
        pltpu.async_copy(hsp_hbm.at[sgi[b]], hbuf[b], sem[b])

    def wait_gather(b):
        pltpu.make_async_copy(hsp_hbm.at[sgi[b]], hbuf[b], sem[b]).wait()

    for k in range(HEADS // NSC):
        fb = c * (HEADS // NSC) + k

        def zrow(r, _):
            for v in range(AW // 16):
                hbuf0[r, pl.ds(v * 16, 16)] = jnp.zeros((16,), jnp.float32)
            return 0
        lax.fori_loop(0, CB, zrow, 0)
        for z in range(STRIPE_A // CB):
            pltpu.sync_copy(hbuf0, acc.at[pl.ds(s * STRIPE_A + z * CB, CB)])
        plsc.subcore_barrier()

        base = s * estripe
        pltpu.sync_copy(dst_hbm.at[pl.ds(base, CB)], didx[0])
        issue_loads(0, base)
        wait_loads(0)
        issue_gather(0, fb)
        pltpu.sync_copy(dst_hbm.at[pl.ds(base + CB, CB)], didx[1])
        issue_loads(1, base + CB)

        def pair(q2, _):
            q = q2 * 2

            def half(b, qa):
                qn1 = jnp.minimum(qa + 1, NCH - 1)
                qn2 = jnp.minimum(qa + 2, NCH - 1)
                wait_loads(1 - b)
                issue_gather(1 - b, fb)
                wait_gather(b)

                @pl.when(c == 0)
                def _():
                    _scale_rows(hbuf[b], atb[b], k)

                @pl.when(c == 1)
                def _():
                    _scale_rows(hbuf[b], atb[b], HEADS // NSC + k)

                issue_loads(b, base + qn2 * CB)
                pltpu.sync_copy(hbuf[b], acc.at[didx[b]], add=True)
                pltpu.async_copy(dst_hbm.at[pl.ds(base + qn2 * CB, CB)],
                                 didx[b], dsem[b])
                pltpu.make_async_copy(dst_hbm.at[pl.ds(0, CB)], didx[b],
                                      dsem[b]).wait()
            half(0, q)
            half(1, q + 1)
            return 0
        lax.fori_loop(0, NCH // 2, pair, 0)
        wait_loads(1)
        wait_gather(0)
        plsc.subcore_barrier()

        pltpu.sync_copy(acc.at[pl.ds(s * STRIPE_A, STRIPE_A)],

                        out_hbm.at[fb, pl.ds(s * STRIPE_A, STRIPE_A)])
        plsc.subcore_barrier()


_agg1_kernel = functools.partial(
    pl.kernel,
    out_type=jax.ShapeDtypeStruct((HEADS, NPAD, AW), jnp.float32),
    mesh=_sc_mesh,
    scratch_types=[
        pltpu.VMEM((CB,), jnp.int32),
        pltpu.VMEM((CB,), jnp.int32),
        pltpu.VMEM((CB,), jnp.int32),
        pltpu.VMEM((CB,), jnp.int32),
        pltpu.VMEM((CB,), jnp.int32),
        pltpu.VMEM((CB,), jnp.int32),
        pltpu.VMEM((CB * 16,), jnp.float32),
        pltpu.VMEM((CB * 16,), jnp.float32),
        pltpu.VMEM((CB, AW), jnp.float32),
        pltpu.VMEM((CB, AW), jnp.float32),
        pltpu.VMEM_SHARED((NPAD, AW), jnp.float32),
        pltpu.SemaphoreType.DMA,
        pltpu.SemaphoreType.DMA,
        pltpu.SemaphoreType.DMA,
        pltpu.SemaphoreType.DMA,
        pltpu.SemaphoreType.DMA,
        pltpu.SemaphoreType.DMA,
    ],
)(_agg1_body)


def _agg2_body(src_hbm, dst_hbm, att_hbm, h_hbm, out_hbm,
               sidx0, sidx1, didx0, didx1, atb0, atb1, hbuf0, hbuf1,
               acc, sem0, sem1):
    c = lax.axis_index("c")
    s = lax.axis_index("s")
    wid = s * NSC + c
    NCH = C_TILE // CB
    sidx = (sidx0, sidx1)
    didx = (didx0, didx1)
    atb = (atb0, atb1)
    hbuf = (hbuf0, hbuf1)
    sem = (sem0, sem1)

    def start_chunk(b, off):
        pltpu.sync_copy(src_hbm.at[pl.ds(off, CB)], sidx[b])
        pltpu.sync_copy(dst_hbm.at[pl.ds(off, CB)], didx[b])
        pltpu.sync_copy(att_hbm.at[pl.ds(off * 16, CB * 16)], atb[b])
        pltpu.async_copy(h_hbm.at[sidx[b]], hbuf[b], sem[b])

    def wait_chunk(b):
        pltpu.make_async_copy(h_hbm.at[sidx[b]], hbuf[b], sem[b]).wait()

    def zrow(r, _):
        for v in range(AW // 16):
            hbuf0[r, pl.ds(v * 16, 16)] = jnp.zeros((16,), jnp.float32)
        return 0
    lax.fori_loop(0, CB, zrow, 0)
    for z in range(STRIPE_A // CB):
        pltpu.sync_copy(hbuf0, acc.at[pl.ds(s * STRIPE_A + z * CB, CB)])
    plsc.subcore_barrier()

    base = wid * C_TILE
    start_chunk(0, base)

    def pair(q2, _):
        q = q2 * 2

        def half(b, qa):
            qn = jnp.minimum(qa + 1, NCH - 1)
            start_chunk(1 - b, base + qn * CB)
            wait_chunk(b)
            _scale_rows(hbuf[b], atb[b], 0)
            pltpu.sync_copy(hbuf[b], acc.at[didx[b]], add=True)
        half(0, q)
        half(1, q + 1)
        return 0
    lax.fori_loop(0, NCH // 2, pair, 0)
    wait_chunk(0)
    plsc.subcore_barrier()

    pltpu.sync_copy(acc.at[pl.ds(s * STRIPE_A, STRIPE_A)],
                    out_hbm.at[c, pl.ds(s * STRIPE_A, STRIPE_A)])


_agg2_kernel = functools.partial(
    pl.kernel,
    out_type=jax.ShapeDtypeStruct((NSC, NPAD, AW), jnp.float32),
    mesh=_sc_mesh,
    scratch_types=(
        [pltpu.VMEM((CB,), jnp.int32)] * 4
        + [pltpu.VMEM((CB * 16,), jnp.float32)] * 2
        + [pltpu.VMEM((CB, AW), jnp.float32)] * 2
        + [pltpu.VMEM_SHARED((NPAD, AW), jnp.float32)]
        + [pltpu.SemaphoreType.DMA] * 2
    ),
)(_agg2_body)


_COMB_BLK = 1024


def _comb_body(al_ref, d0_ref, d1_ref, dd_ref):
    colmask = lax.broadcasted_iota(jnp.int32, (_COMB_BLK, AW), 1) < TW
    dd_ref[...] = jnp.where(colmask, d0_ref[...] + d1_ref[...], al_ref[...])


def _combine(alp, d0, d1):
    return pl.pallas_call(
        _comb_body,
        grid=(NPAD // _COMB_BLK,),
        in_specs=[pl.BlockSpec((_COMB_BLK, AW), lambda i: (i, 0))] * 3,
        out_specs=pl.BlockSpec((_COMB_BLK, AW), lambda i: (i, 0)),
        out_shape=jax.ShapeDtypeStruct((NPAD, AW), jnp.float32),
    )(alp, d0, d1)


def _mm1_body(x_ref, w_ref, am_ref, h_ref, al_ref):
    h = jnp.dot(x_ref[...], w_ref[...], preferred_element_type=jnp.float32)
    h_ref[...] = h
    al_ref[...] = jnp.dot(h, am_ref[...], preferred_element_type=jnp.float32)


def _mm1(x, W, a_mat, d_in, d_out):
    grid = N // ROW_BLK
    return pl.pallas_call(
        _mm1_body,
        grid=(grid,),
        in_specs=[
            pl.BlockSpec((ROW_BLK, d_in), lambda i: (i, 0)),
            pl.BlockSpec((d_in, d_out), lambda i: (0, 0)),
            pl.BlockSpec((d_out, AW), lambda i: (0, 0)),
        ],
        out_specs=[
            pl.BlockSpec((ROW_BLK, d_out), lambda i: (i, 0)),
            pl.BlockSpec((ROW_BLK, AW), lambda i: (i, 0)),
        ],
        out_shape=[
            jax.ShapeDtypeStruct((N, d_out), jnp.float32),
            jax.ShapeDtypeStruct((N, AW), jnp.float32),
        ],
    )(x, W, a_mat)


def _mid_body(agg_ref, b1_ref, w2_ref, am_ref, g_ref, al_ref):
    h1 = jnp.maximum(agg_ref[...] + b1_ref[...], 0.0)
    g = jnp.dot(h1, w2_ref[...], preferred_element_type=jnp.float32)
    g_ref[...] = g
    al_ref[...] = jnp.dot(g, am_ref[...], preferred_element_type=jnp.float32)


def _mid(agg1, b1, W2, a_mat2):
    grid = N // ROW_BLK
    return pl.pallas_call(
        _mid_body,
        grid=(grid,),
        in_specs=[
            pl.BlockSpec((ROW_BLK, HEADS * D), lambda i: (i, 0)),
            pl.BlockSpec((1, HEADS * D), lambda i: (0, 0)),
            pl.BlockSpec((HEADS * D, OUT), lambda i: (0, 0)),
            pl.BlockSpec((OUT, AW), lambda i: (0, 0)),
        ],
        out_specs=[
            pl.BlockSpec((ROW_BLK, OUT), lambda i: (i, 0)),
            pl.BlockSpec((ROW_BLK, AW), lambda i: (i, 0)),
        ],
        out_shape=[
            jax.ShapeDtypeStruct((N, OUT), jnp.float32),
            jax.ShapeDtypeStruct((N, AW), jnp.float32),
        ],
    )(agg1, b1.reshape(1, HEADS * D), W2, a_mat2)


def _fin_body(p0_ref, p1_ref, b2_ref, fcw_ref, fcb_ref, y_ref, gr_ref, mx_ref):
    i = pl.program_id(0)
    h2 = jnp.maximum(p0_ref[...] + p1_ref[...] + b2_ref[...], 0.0)
    y_ref[...] = (
        jnp.dot(h2, fcw_ref[...], preferred_element_type=jnp.float32)
        + fcb_ref[...]
    )
    blockmax = jnp.max(h2, axis=0, keepdims=True)

    @pl.when(i == 0)
    def _():
        mx_ref[...] = blockmax

    @pl.when(i > 0)
    def _():
        mx_ref[...] = jnp.maximum(mx_ref[...], blockmax)

    @pl.when(i == pl.num_programs(0) - 1)
    def _():
        gr_ref[...] = (
            jnp.dot(mx_ref[...], fcw_ref[...], preferred_element_type=jnp.float32)
            + fcb_ref[...]
        )


def _fin(p0, p1, b2, fcW, fcb):
    grid = N // ROW_BLK
    return pl.pallas_call(
        _fin_body,
        grid=(grid,),
        in_specs=[
            pl.BlockSpec((ROW_BLK, OUT), lambda i: (i, 0)),
            pl.BlockSpec((ROW_BLK, OUT), lambda i: (i, 0)),
            pl.BlockSpec((1, OUT), lambda i: (0, 0)),
            pl.BlockSpec((OUT, OUT), lambda i: (0, 0)),
            pl.BlockSpec((1, OUT), lambda i: (0, 0)),
        ],
        out_specs=[
            pl.BlockSpec((ROW_BLK, OUT), lambda i: (i, 0)),
            pl.BlockSpec((1, OUT), lambda i: (0, 0)),
        ],
        out_shape=[
            jax.ShapeDtypeStruct((N, OUT), jnp.float32),
            jax.ShapeDtypeStruct((1, OUT), jnp.float32),
        ],
        scratch_shapes=[pltpu.VMEM((1, OUT), jnp.float32)],
    )(p0, p1, b2.reshape(1, OUT), fcW, fcb.reshape(1, OUT))


def _pack_alpha(a_src, a_dst, d_out, heads, head_dim):
    m = jnp.zeros((d_out, AW), jnp.float32)
    for h in range(heads):
        m = m.at[h * head_dim:(h + 1) * head_dim, h].set(a_src[h])
        m = m.at[h * head_dim:(h + 1) * head_dim, TW + h].set(a_dst[h])
    return m


def kernel(x, edge_index, graph_id, W1, a_src1, a_dst1, b1, W2, a_src2,
           a_dst2, b2, fcW, fcb):
    loops = jnp.arange(N, dtype=edge_index.dtype)
    pad = jnp.zeros((EP - E - N,), jnp.int32)
    src = jnp.concatenate([edge_index[0], loops])
    dst = jnp.concatenate([edge_index[1], loops])
    src_p = jnp.concatenate([src, pad])
    dst_p = jnp.concatenate([dst, pad + N])

    a_mat1 = _pack_alpha(a_src1, a_dst1, HEADS * D, HEADS, D)
    a_mat2 = _pack_alpha(a_src2, a_dst2, OUT, 1, OUT)

    def tables(al):
        return jnp.pad(al, ((0, NPAD - N), (0, 0)))

    h1mm, al1 = _mm1(x, W1, a_mat1, D, HEADS * D)
    al1p = tables(al1)
    dns1 = _denom_kernel(src_p, dst_p, al1p)
    dd1 = _combine(al1p, dns1[0], dns1[1])
    att1 = _att_kernel(src_p, dst_p, al1p, dd1)
    hsp = h1mm.reshape(N, HEADS, D).swapaxes(0, 1).reshape(HEADS * N, D)
    out1 = _agg1_kernel(src_p, dst_p, att1, hsp)
    agg1 = out1[:, :N, :].swapaxes(0, 1).reshape(N, HEADS * D)
    g2, al2 = _mid(agg1, b1, W2, a_mat2)
    al2p = tables(al2)
    dns2 = _denom_kernel(src_p, dst_p, al2p)
    dd2 = _combine(al2p, dns2[0], dns2[1])
    att2 = _att_kernel(src_p, dst_p, al2p, dd2)
    agg2p = _agg2_kernel(src_p, dst_p, att2, g2)
    y, gr = _fin(agg2p[0, :N], agg2p[1, :N], b2, fcW, fcb)
    return y[None, :, :], gr


# speedup vs baseline: 9.8556x; 1.0533x over previous
import functools

import jax
import jax.numpy as jnp
import numpy as np
from jax import lax
from jax.experimental import pallas as pl
from jax.experimental.pallas import tpu as pltpu
from jax.experimental.pallas import tpu_sc as plsc

N = 10000
E = 160000
D = 128
HEADS = 10
OUT = 128
NEG_SLOPE = 0.2
TW = 16

ROW_BLK = 1000

NSC = 2
NTILES = 16
EP = 172032
C_TILE = EP // (NSC * NTILES)
CB = 128
NPAD = 10240
STRIPE_A = NPAD // NTILES
AW = 128

_sc_mesh = plsc.VectorSubcoreMesh(core_axis_name="c", subcore_axis_name="s")


DCB = 64


def _denom_body(src_hbm, dst_hbm, al_hbm, out_hbm,
                sidx0, sidx1, didx0, didx1, esb0, esb1, edb0, edb1,
                acc, gs0, gs1, gd0, gd1):
    c = lax.axis_index("c")
    s = lax.axis_index("s")
    wid = s * NSC + c
    NCH = C_TILE // DCB
    sidx = (sidx0, sidx1)
    didx = (didx0, didx1)
    esb = (esb0, esb1)
    edb = (edb0, edb1)
    gs = (gs0, gs1)
    gd = (gd0, gd1)

    def zrow(i, _):
        for j in range(AW // 16):
            edb0[i, pl.ds(j * 16, 16)] = jnp.zeros((16,), jnp.float32)
        return 0
    lax.fori_loop(0, DCB, zrow, 0)
    for z in range(STRIPE_A // DCB):
        pltpu.sync_copy(edb0, acc.at[pl.ds(s * STRIPE_A + z * DCB, DCB)])
    plsc.subcore_barrier()

    def start_chunk(b, off):
        pltpu.sync_copy(src_hbm.at[pl.ds(off, DCB)], sidx[b])
        pltpu.sync_copy(dst_hbm.at[pl.ds(off, DCB)], didx[b])
        pltpu.async_copy(al_hbm.at[sidx[b]], esb[b], gs[b])
        pltpu.async_copy(al_hbm.at[didx[b]], edb[b], gd[b])

    def wait_chunk(b):
        pltpu.make_async_copy(al_hbm.at[sidx[b]], esb[b], gs[b]).wait()
        pltpu.make_async_copy(al_hbm.at[didx[b]], edb[b], gd[b]).wait()

    base = wid * C_TILE
    start_chunk(0, base)

    def pair(q2, _):
        q = q2 * 2

        def half(b, qa):
            qn = jnp.minimum(qa + 1, NCH - 1)
            start_chunk(1 - b, base + qn * DCB)
            wait_chunk(b)

            def row(j, _):
                e = esb[b][j, pl.ds(0, 16)] + edb[b][j, pl.ds(16, 16)]
                e = jnp.where(e >= 0.0, e, NEG_SLOPE * e)
                edb[b][j, pl.ds(0, 16)] = jnp.exp(e)
                return 0
            lax.fori_loop(0, DCB, row, 0, unroll=4)
            pltpu.sync_copy(edb[b], acc.at[didx[b]], add=True)
        half(0, q)
        half(1, q + 1)
        return 0
    lax.fori_loop(0, NCH // 2, pair, 0)
    wait_chunk(0)
    plsc.subcore_barrier()

    pltpu.sync_copy(acc.at[pl.ds(s * STRIPE_A, STRIPE_A)],
                    out_hbm.at[c, pl.ds(s * STRIPE_A, STRIPE_A)])


_denom_kernel = functools.partial(
    pl.kernel,
    out_type=jax.ShapeDtypeStruct((NSC, NPAD, AW), jnp.float32),
    mesh=_sc_mesh,
    scratch_types=(
        [pltpu.VMEM((DCB,), jnp.int32)] * 4
        + [pltpu.VMEM((DCB, AW), jnp.float32)] * 4
        + [pltpu.VMEM_SHARED((NPAD, AW), jnp.float32)]
        + [pltpu.SemaphoreType.DMA] * 4
    ),
)(_denom_body)


def _att_body(src_hbm, dst_hbm, al_hbm, dd_hbm, att_hbm,
              sidx0, sidx1, didx0, didx1, esb0, esb1, ddb0, ddb1,
              atb0, atb1, gs0, gs1, gd0, gd1):
    c = lax.axis_index("c")
    s = lax.axis_index("s")
    wid = s * NSC + c
    NCH = C_TILE // CB
    sidx = (sidx0, sidx1)
    didx = (didx0, didx1)
    esb = (esb0, esb1)
    ddb = (ddb0, ddb1)
    atb = (atb0, atb1)
    gs = (gs0, gs1)
    gd = (gd0, gd1)

    def start_chunk(b, off):
        pltpu.sync_copy(src_hbm.at[pl.ds(off, CB)], sidx[b])
        pltpu.sync_copy(dst_hbm.at[pl.ds(off, CB)], didx[b])
        pltpu.async_copy(al_hbm.at[sidx[b]], esb[b], gs[b])
        pltpu.async_copy(dd_hbm.at[didx[b]], ddb[b], gd[b])

    def wait_chunk(b):
        pltpu.make_async_copy(al_hbm.at[sidx[b]], esb[b], gs[b]).wait()
        pltpu.make_async_copy(dd_hbm.at[didx[b]], ddb[b], gd[b]).wait()

    base = wid * C_TILE
    start_chunk(0, base)

    def pair(q2, _):
        q = q2 * 2

        def half(b, qa):
            qn = jnp.minimum(qa + 1, NCH - 1)
            start_chunk(1 - b, base + qn * CB)
            wait_chunk(b)

            def row(j, _):
                e = esb[b][j, pl.ds(0, 16)] + ddb[b][j, pl.ds(16, 16)]
                e = jnp.where(e >= 0.0, e, NEG_SLOPE * e)
                atb[b][pl.ds(j * 16, 16)] = (
                    jnp.exp(e) / ddb[b][j, pl.ds(0, 16)])
                return 0
            lax.fori_loop(0, CB, row, 0, unroll=4)
            pltpu.sync_copy(atb[b],
                            att_hbm.at[pl.ds((base + qa * CB) * 16, CB * 16)])
        half(0, q)
        half(1, q + 1)
        return 0
    lax.fori_loop(0, NCH // 2, pair, 0)
    wait_chunk(0)


_att_kernel = functools.partial(
    pl.kernel,
    out_type=jax.ShapeDtypeStruct((EP * 16,), jnp.float32),
    mesh=_sc_mesh,
    scratch_types=(
        [pltpu.VMEM((CB,), jnp.int32)] * 4
        + [pltpu.VMEM((CB, AW), jnp.float32)] * 4
        + [pltpu.VMEM((CB * 16,), jnp.float32)] * 2
        + [pltpu.SemaphoreType.DMA] * 4
    ),
)(_att_body)


def _scale_rows(hbuf, atb, lane):
    def srow(r, _):
        av = atb[pl.ds(r * 16, 16)]
        a = av[lane]
        for v in range(AW // 16):
            hbuf[r, pl.ds(v * 16, 16)] = hbuf[r, pl.ds(v * 16, 16)] * a
        return 0
    lax.fori_loop(0, CB, srow, 0, unroll=4)


def _agg1_body(src_hbm, dst_hbm, att_hbm, hsp_hbm, out_hbm,
               sidx0, sidx1, didx0, didx1, sgi0, sgi1, atb0, atb1,
               hbuf0, hbuf1, acc, sem0, sem1, lsem0, lsem1, dsem0, dsem1):
    c = lax.axis_index("c")
    s = lax.axis_index("s")
    estripe = EP // NTILES
    NCH = estripe // CB
    sidx = (sidx0, sidx1)
    didx = (didx0, didx1)
    sgi = (sgi0, sgi1)
    atb = (atb0, atb1)
    hbuf = (hbuf0, hbuf1)
    sem = (sem0, sem1)
    lsem = (lsem0, lsem1)
    dsem = (dsem0, dsem1)

    def issue_loads(b, off):
        pltpu.async_copy(src_hbm.at[pl.ds(off, CB)], sidx[b], lsem[b])
        pltpu.async_copy(att_hbm.at[pl.ds(off * 16, CB * 16)], atb[b], lsem[b])

    def wait_loads(b):
        pltpu.make_async_copy(src_hbm.at[pl.ds(0, CB)], sidx[b], lsem[b]).wait()
        pltpu.make_async_copy(att_hbm.at[pl.ds(0, CB * 16)], atb[b], lsem[b]).wait()

    def issue_gather(b, fb):
        def addoff(j, _):
            sgi[b][pl.ds(j * 16, 16)] = sidx[b][pl.ds(j * 16, 16)] + fb * N
            return 0
        lax.fori_loop(0, CB // 16, addoff, 0)
        pltpu.async_copy(hsp_hbm.at[sgi[b]], hbuf[b], sem[b])

    def wait_gather(b):
        pltpu.make_async_copy(hsp_hbm.at[sgi[b]], hbuf[b], sem[b]).wait()

    for k in range(HEADS // NSC):
        fb = c * (HEADS // NSC) + k

        def zrow(r, _):
            for v in range(AW // 16):
                hbuf0[r, pl.ds(v * 16, 16)] = jnp.zeros((16,), jnp.float32)
            return 0
        lax.fori_loop(0, CB, zrow, 0)
        for z in range(STRIPE_A // CB):
            pltpu.sync_copy(hbuf0, acc.at[pl.ds(s * STRIPE_A + z * CB, CB)])
        plsc.subcore_barrier()

        base = s * estripe
        pltpu.sync_copy(dst_hbm.at[pl.ds(base, CB)], didx[0])
        issue_loads(0, base)
        wait_loads(0)
        issue_gather(0, fb)
        pltpu.sync_copy(dst_hbm.at[pl.ds(base + CB, CB)], didx[1])
        issue_loads(1, base + CB)

        def pair(q2, _):
            q = q2 * 2

            def half(b, qa):
                qn1 = jnp.minimum(qa + 1, NCH - 1)
                qn2 = jnp.minimum(qa + 2, NCH - 1)
                wait_loads(1 - b)
                issue_gather(1 - b, fb)
                wait_gather(b)

                @pl.when(c == 0)
                def _():
                    _scale_rows(hbuf[b], atb[b], k)

                @pl.when(c == 1)
                def _():
                    _scale_rows(hbuf[b], atb[b], HEADS // NSC + k)

                issue_loads(b, base + qn2 * CB)
                pltpu.sync_copy(hbuf[b], acc.at[didx[b]], add=True)
                pltpu.async_copy(dst_hbm.at[pl.ds(base + qn2 * CB, CB)],
                                 didx[b], dsem[b])
                pltpu.make_async_copy(dst_hbm.at[pl.ds(0, CB)], didx[b],
                                      dsem[b]).wait()
            half(0, q)
            half(1, q + 1)
            return 0
        lax.fori_loop(0, NCH // 2, pair, 0)
        wait_loads(1)
        wait_gather(0)
        plsc.subcore_barrier()

        pltpu.sync_copy(acc.at[pl.ds(s * STRIPE_A, STRIPE_A)],

                        out_hbm.at[fb, pl.ds(s * STRIPE_A, STRIPE_A)])
        plsc.subcore_barrier()


_agg1_kernel = functools.partial(
    pl.kernel,
    out_type=jax.ShapeDtypeStruct((HEADS, NPAD, AW), jnp.float32),
    mesh=_sc_mesh,
    scratch_types=[
        pltpu.VMEM((CB,), jnp.int32),
        pltpu.VMEM((CB,), jnp.int32),
        pltpu.VMEM((CB,), jnp.int32),
        pltpu.VMEM((CB,), jnp.int32),
        pltpu.VMEM((CB,), jnp.int32),
        pltpu.VMEM((CB,), jnp.int32),
        pltpu.VMEM((CB * 16,), jnp.float32),
        pltpu.VMEM((CB * 16,), jnp.float32),
        pltpu.VMEM((CB, AW), jnp.float32),
        pltpu.VMEM((CB, AW), jnp.float32),
        pltpu.VMEM_SHARED((NPAD, AW), jnp.float32),
        pltpu.SemaphoreType.DMA,
        pltpu.SemaphoreType.DMA,
        pltpu.SemaphoreType.DMA,
        pltpu.SemaphoreType.DMA,
        pltpu.SemaphoreType.DMA,
        pltpu.SemaphoreType.DMA,
    ],
)(_agg1_body)


def _agg2_body(src_hbm, dst_hbm, att_hbm, h_hbm, out_hbm,
               sidx0, sidx1, didx0, didx1, atb0, atb1, hbuf0, hbuf1,
               acc, sem0, sem1):
    c = lax.axis_index("c")
    s = lax.axis_index("s")
    wid = s * NSC + c
    NCH = C_TILE // CB
    sidx = (sidx0, sidx1)
    didx = (didx0, didx1)
    atb = (atb0, atb1)
    hbuf = (hbuf0, hbuf1)
    sem = (sem0, sem1)

    def start_chunk(b, off):
        pltpu.sync_copy(src_hbm.at[pl.ds(off, CB)], sidx[b])
        pltpu.sync_copy(dst_hbm.at[pl.ds(off, CB)], didx[b])
        pltpu.sync_copy(att_hbm.at[pl.ds(off * 16, CB * 16)], atb[b])
        pltpu.async_copy(h_hbm.at[sidx[b]], hbuf[b], sem[b])

    def wait_chunk(b):
        pltpu.make_async_copy(h_hbm.at[sidx[b]], hbuf[b], sem[b]).wait()

    def zrow(r, _):
        for v in range(AW // 16):
            hbuf0[r, pl.ds(v * 16, 16)] = jnp.zeros((16,), jnp.float32)
        return 0
    lax.fori_loop(0, CB, zrow, 0)
    for z in range(STRIPE_A // CB):
        pltpu.sync_copy(hbuf0, acc.at[pl.ds(s * STRIPE_A + z * CB, CB)])
    plsc.subcore_barrier()

    base = wid * C_TILE
    start_chunk(0, base)

    def pair(q2, _):
        q = q2 * 2

        def half(b, qa):
            qn = jnp.minimum(qa + 1, NCH - 1)
            start_chunk(1 - b, base + qn * CB)
            wait_chunk(b)
            _scale_rows(hbuf[b], atb[b], 0)
            pltpu.sync_copy(hbuf[b], acc.at[didx[b]], add=True)
        half(0, q)
        half(1, q + 1)
        return 0
    lax.fori_loop(0, NCH // 2, pair, 0)
    wait_chunk(0)
    plsc.subcore_barrier()

    pltpu.sync_copy(acc.at[pl.ds(s * STRIPE_A, STRIPE_A)],
                    out_hbm.at[c, pl.ds(s * STRIPE_A, STRIPE_A)])


_agg2_kernel = functools.partial(
    pl.kernel,
    out_type=jax.ShapeDtypeStruct((NSC, NPAD, AW), jnp.float32),
    mesh=_sc_mesh,
    scratch_types=(
        [pltpu.VMEM((CB,), jnp.int32)] * 4
        + [pltpu.VMEM((CB * 16,), jnp.float32)] * 2
        + [pltpu.VMEM((CB, AW), jnp.float32)] * 2
        + [pltpu.VMEM_SHARED((NPAD, AW), jnp.float32)]
        + [pltpu.SemaphoreType.DMA] * 2
    ),
)(_agg2_body)


_COMB_BLK = 1024


def _comb_body(al_ref, d0_ref, d1_ref, dd_ref):
    colmask = lax.broadcasted_iota(jnp.int32, (_COMB_BLK, AW), 1) < TW
    dd_ref[...] = jnp.where(colmask, d0_ref[...] + d1_ref[...], al_ref[...])


def _combine(alp, d0, d1):
    return pl.pallas_call(
        _comb_body,
        grid=(NPAD // _COMB_BLK,),
        in_specs=[pl.BlockSpec((_COMB_BLK, AW), lambda i: (i, 0))] * 3,
        out_specs=pl.BlockSpec((_COMB_BLK, AW), lambda i: (i, 0)),
        out_shape=jax.ShapeDtypeStruct((NPAD, AW), jnp.float32),
    )(alp, d0, d1)


def _mm1_body(x_ref, w_ref, am_ref, h_ref, al_ref):
    h = jnp.dot(x_ref[...], w_ref[...], preferred_element_type=jnp.float32)
    h_ref[...] = h
    al_ref[...] = jnp.dot(h, am_ref[...], preferred_element_type=jnp.float32)


def _mm1(x, W, a_mat, d_in, d_out):
    grid = N // ROW_BLK
    return pl.pallas_call(
        _mm1_body,
        grid=(grid,),
        in_specs=[
            pl.BlockSpec((ROW_BLK, d_in), lambda i: (i, 0)),
            pl.BlockSpec((d_in, d_out), lambda i: (0, 0)),
            pl.BlockSpec((d_out, AW), lambda i: (0, 0)),
        ],
        out_specs=[
            pl.BlockSpec((ROW_BLK, d_out), lambda i: (i, 0)),
            pl.BlockSpec((ROW_BLK, AW), lambda i: (i, 0)),
        ],
        out_shape=[
            jax.ShapeDtypeStruct((N, d_out), jnp.float32),
            jax.ShapeDtypeStruct((N, AW), jnp.float32),
        ],
    )(x, W, a_mat)


def _mid_body(agg_ref, b1_ref, w2_ref, am_ref, g_ref, al_ref):
    h1 = jnp.maximum(agg_ref[...] + b1_ref[...], 0.0)
    g = jnp.dot(h1, w2_ref[...], preferred_element_type=jnp.float32)
    g_ref[...] = g
    al_ref[...] = jnp.dot(g, am_ref[...], preferred_element_type=jnp.float32)


def _mid(agg1, b1, W2, a_mat2):
    grid = N // ROW_BLK
    return pl.pallas_call(
        _mid_body,
        grid=(grid,),
        in_specs=[
            pl.BlockSpec((ROW_BLK, HEADS * D), lambda i: (i, 0)),
            pl.BlockSpec((1, HEADS * D), lambda i: (0, 0)),
            pl.BlockSpec((HEADS * D, OUT), lambda i: (0, 0)),
            pl.BlockSpec((OUT, AW), lambda i: (0, 0)),
        ],
        out_specs=[
            pl.BlockSpec((ROW_BLK, OUT), lambda i: (i, 0)),
            pl.BlockSpec((ROW_BLK, AW), lambda i: (i, 0)),
        ],
        out_shape=[
            jax.ShapeDtypeStruct((N, OUT), jnp.float32),
            jax.ShapeDtypeStruct((N, AW), jnp.float32),
        ],
    )(agg1, b1.reshape(1, HEADS * D), W2, a_mat2)


def _fin_body(p0_ref, p1_ref, b2_ref, fcw_ref, fcb_ref, y_ref, gr_ref, mx_ref):
    i = pl.program_id(0)
    h2 = jnp.maximum(p0_ref[...] + p1_ref[...] + b2_ref[...], 0.0)
    y_ref[...] = (
        jnp.dot(h2, fcw_ref[...], preferred_element_type=jnp.float32)
        + fcb_ref[...]
    )
    blockmax = jnp.max(h2, axis=0, keepdims=True)

    @pl.when(i == 0)
    def _():
        mx_ref[...] = blockmax

    @pl.when(i > 0)
    def _():
        mx_ref[...] = jnp.maximum(mx_ref[...], blockmax)

    @pl.when(i == pl.num_programs(0) - 1)
    def _():
        gr_ref[...] = (
            jnp.dot(mx_ref[...], fcw_ref[...], preferred_element_type=jnp.float32)
            + fcb_ref[...]
        )


def _fin(p0, p1, b2, fcW, fcb):
    grid = N // ROW_BLK
    return pl.pallas_call(
        _fin_body,
        grid=(grid,),
        in_specs=[
            pl.BlockSpec((ROW_BLK, OUT), lambda i: (i, 0)),
            pl.BlockSpec((ROW_BLK, OUT), lambda i: (i, 0)),
            pl.BlockSpec((1, OUT), lambda i: (0, 0)),
            pl.BlockSpec((OUT, OUT), lambda i: (0, 0)),
            pl.BlockSpec((1, OUT), lambda i: (0, 0)),
        ],
        out_specs=[
            pl.BlockSpec((ROW_BLK, OUT), lambda i: (i, 0)),
            pl.BlockSpec((1, OUT), lambda i: (0, 0)),
        ],
        out_shape=[
            jax.ShapeDtypeStruct((N, OUT), jnp.float32),
            jax.ShapeDtypeStruct((1, OUT), jnp.float32),
        ],
        scratch_shapes=[pltpu.VMEM((1, OUT), jnp.float32)],
    )(p0, p1, b2.reshape(1, OUT), fcW, fcb.reshape(1, OUT))


def _pack_alpha(a_src, a_dst, d_out, heads, head_dim):
    m = jnp.zeros((d_out, AW), jnp.float32)
    for h in range(heads):
        m = m.at[h * head_dim:(h + 1) * head_dim, h].set(a_src[h])
        m = m.at[h * head_dim:(h + 1) * head_dim, TW + h].set(a_dst[h])
    return m


def kernel(x, edge_index, graph_id, W1, a_src1, a_dst1, b1, W2, a_src2,
           a_dst2, b2, fcW, fcb):
    loops = jnp.arange(N, dtype=edge_index.dtype)
    pad = jnp.zeros((EP - E - N,), jnp.int32)
    src = jnp.concatenate([edge_index[0], loops])
    dst = jnp.concatenate([edge_index[1], loops])
    src_p = jnp.concatenate([src, pad])
    dst_p = jnp.concatenate([dst, pad + N])

    a_mat1 = _pack_alpha(a_src1, a_dst1, HEADS * D, HEADS, D)
    a_mat2 = _pack_alpha(a_src2, a_dst2, OUT, 1, OUT)

    def tables(al):
        return jnp.pad(al, ((0, NPAD - N), (0, 0)))

    h1mm, al1 = _mm1(x, W1, a_mat1, D, HEADS * D)
    al1p = tables(al1)
    dns1 = _denom_kernel(src_p, dst_p, al1p)
    dd1 = _combine(al1p, dns1[0], dns1[1])
    att1 = _att_kernel(src_p, dst_p, al1p, dd1)
    hsp = h1mm.reshape(N, HEADS, D).swapaxes(0, 1).reshape(HEADS * N, D)
    out1 = _agg1_kernel(src_p, dst_p, att1, hsp)
    agg1 = out1[:, :N, :].swapaxes(0, 1).reshape(N, HEADS * D)
    g2, al2 = _mid(agg1, b1, W2, a_mat2)
    al2p = tables(al2)
    dns2 = _denom_kernel(src_p, dst_p, al2p)
    dd2 = _combine(al2p, dns2[0], dns2[1])
    att2 = _att_kernel(src_p, dst_p, al2p, dd2)
    agg2p = _agg2_kernel(src_p, dst_p, att2, g2)
    y, gr = _fin(agg2p[0, :N], agg2p[1, :N], b2, fcW, fcb)
    return y[None, :, :], gr


# speedup vs baseline: 10.3194x; 1.0471x over previous
import functools

import jax
import jax.numpy as jnp
import numpy as np
from jax import lax
from jax.experimental import pallas as pl
from jax.experimental.pallas import tpu as pltpu
from jax.experimental.pallas import tpu_sc as plsc

N = 10000
E = 160000
D = 128
HEADS = 10
OUT = 128
NEG_SLOPE = 0.2
TW = 16

ROW_BLK = 1000

NSC = 2
NTILES = 16
EP = 172032
C_TILE = EP // (NSC * NTILES)
CB = 128
NPAD = 10240
STRIPE_A = NPAD // NTILES
AW = 128

_sc_mesh = plsc.VectorSubcoreMesh(core_axis_name="c", subcore_axis_name="s")


DCB = 64


def _denom_body(src_hbm, dst_hbm, al_hbm, out_hbm,
                sidx0, sidx1, didx0, didx1, esb0, esb1, edb0, edb1,
                acc, gs0, gs1, gd0, gd1):
    c = lax.axis_index("c")
    s = lax.axis_index("s")
    wid = s * NSC + c
    NCH = C_TILE // DCB
    sidx = (sidx0, sidx1)
    didx = (didx0, didx1)
    esb = (esb0, esb1)
    edb = (edb0, edb1)
    gs = (gs0, gs1)
    gd = (gd0, gd1)

    def zrow(i, _):
        for j in range(AW // 16):
            edb0[i, pl.ds(j * 16, 16)] = jnp.zeros((16,), jnp.float32)
        return 0
    lax.fori_loop(0, DCB, zrow, 0)
    for z in range(STRIPE_A // DCB):
        pltpu.sync_copy(edb0, acc.at[pl.ds(s * STRIPE_A + z * DCB, DCB)])
    plsc.subcore_barrier()

    def start_chunk(b, off):
        pltpu.sync_copy(src_hbm.at[pl.ds(off, DCB)], sidx[b])
        pltpu.sync_copy(dst_hbm.at[pl.ds(off, DCB)], didx[b])
        pltpu.async_copy(al_hbm.at[sidx[b]], esb[b], gs[b])
        pltpu.async_copy(al_hbm.at[didx[b]], edb[b], gd[b])

    def wait_chunk(b):
        pltpu.make_async_copy(al_hbm.at[sidx[b]], esb[b], gs[b]).wait()
        pltpu.make_async_copy(al_hbm.at[didx[b]], edb[b], gd[b]).wait()

    base = wid * C_TILE
    start_chunk(0, base)

    def pair(q2, _):
        q = q2 * 2

        def half(b, qa):
            qn = jnp.minimum(qa + 1, NCH - 1)
            start_chunk(1 - b, base + qn * DCB)
            wait_chunk(b)

            def row(j, _):
                e = esb[b][j, pl.ds(0, 16)] + edb[b][j, pl.ds(16, 16)]
                e = jnp.where(e >= 0.0, e, NEG_SLOPE * e)
                edb[b][j, pl.ds(0, 16)] = jnp.exp(e)
                return 0
            lax.fori_loop(0, DCB, row, 0, unroll=4)
            pltpu.sync_copy(edb[b], acc.at[didx[b]], add=True)
        half(0, q)
        half(1, q + 1)
        return 0
    lax.fori_loop(0, NCH // 2, pair, 0)
    wait_chunk(0)
    plsc.subcore_barrier()

    pltpu.sync_copy(acc.at[pl.ds(s * STRIPE_A, STRIPE_A)],
                    out_hbm.at[c, pl.ds(s * STRIPE_A, STRIPE_A)])


_denom_kernel = functools.partial(
    pl.kernel,
    out_type=jax.ShapeDtypeStruct((NSC, NPAD, AW), jnp.float32),
    mesh=_sc_mesh,
    scratch_types=(
        [pltpu.VMEM((DCB,), jnp.int32)] * 4
        + [pltpu.VMEM((DCB, AW), jnp.float32)] * 4
        + [pltpu.VMEM_SHARED((NPAD, AW), jnp.float32)]
        + [pltpu.SemaphoreType.DMA] * 4
    ),
)(_denom_body)


def _att_body(src_hbm, dst_hbm, al_hbm, dd_hbm, att_hbm,
              sidx0, sidx1, didx0, didx1, esb0, esb1, ddb0, ddb1,
              atb0, atb1, gs0, gs1, gd0, gd1):
    c = lax.axis_index("c")
    s = lax.axis_index("s")
    wid = s * NSC + c
    NCH = C_TILE // CB
    sidx = (sidx0, sidx1)
    didx = (didx0, didx1)
    esb = (esb0, esb1)
    ddb = (ddb0, ddb1)
    atb = (atb0, atb1)
    gs = (gs0, gs1)
    gd = (gd0, gd1)

    def start_chunk(b, off):
        pltpu.sync_copy(src_hbm.at[pl.ds(off, CB)], sidx[b])
        pltpu.sync_copy(dst_hbm.at[pl.ds(off, CB)], didx[b])
        pltpu.async_copy(al_hbm.at[sidx[b]], esb[b], gs[b])
        pltpu.async_copy(dd_hbm.at[didx[b]], ddb[b], gd[b])

    def wait_chunk(b):
        pltpu.make_async_copy(al_hbm.at[sidx[b]], esb[b], gs[b]).wait()
        pltpu.make_async_copy(dd_hbm.at[didx[b]], ddb[b], gd[b]).wait()

    base = wid * C_TILE
    start_chunk(0, base)

    def pair(q2, _):
        q = q2 * 2

        def half(b, qa):
            qn = jnp.minimum(qa + 1, NCH - 1)
            start_chunk(1 - b, base + qn * CB)
            wait_chunk(b)

            def row(j, _):
                e = esb[b][j, pl.ds(0, 16)] + ddb[b][j, pl.ds(16, 16)]
                e = jnp.where(e >= 0.0, e, NEG_SLOPE * e)
                atb[b][pl.ds(j * 16, 16)] = (
                    jnp.exp(e) / ddb[b][j, pl.ds(0, 16)])
                return 0
            lax.fori_loop(0, CB, row, 0, unroll=4)
            pltpu.sync_copy(atb[b],
                            att_hbm.at[pl.ds((base + qa * CB) * 16, CB * 16)])
        half(0, q)
        half(1, q + 1)
        return 0
    lax.fori_loop(0, NCH // 2, pair, 0)
    wait_chunk(0)


_att_kernel = functools.partial(
    pl.kernel,
    out_type=jax.ShapeDtypeStruct((EP * 16,), jnp.float32),
    mesh=_sc_mesh,
    scratch_types=(
        [pltpu.VMEM((CB,), jnp.int32)] * 4
        + [pltpu.VMEM((CB, AW), jnp.float32)] * 4
        + [pltpu.VMEM((CB * 16,), jnp.float32)] * 2
        + [pltpu.SemaphoreType.DMA] * 4
    ),
)(_att_body)


def _scale_rows(hbuf, atb, lane):
    def srow(r, _):
        av = atb[pl.ds(r * 16, 16)]
        a = av[lane]
        for v in range(AW // 16):
            hbuf[r, pl.ds(v * 16, 16)] = hbuf[r, pl.ds(v * 16, 16)] * a
        return 0
    lax.fori_loop(0, CB, srow, 0, unroll=4)


def _agg1_body(src_hbm, dst_hbm, att_hbm, hsp_hbm, out_hbm,
               sidx0, sidx1, didx0, didx1, sgi0, sgi1, atb0, atb1,
               hbuf0, hbuf1, dsc0, dsc1, acc, sem0, sem1, lsem0, lsem1,
               dsem0, dsem1, ssem0, ssem1):
    c = lax.axis_index("c")
    s = lax.axis_index("s")
    estripe = EP // NTILES
    NCH = estripe // CB
    sidx = (sidx0, sidx1)
    didx = (didx0, didx1)
    sgi = (sgi0, sgi1)
    atb = (atb0, atb1)
    hbuf = (hbuf0, hbuf1)
    sem = (sem0, sem1)
    lsem = (lsem0, lsem1)
    dsem = (dsem0, dsem1)
    ssem = (ssem0, ssem1)
    dsc = (dsc0, dsc1)

    def wait_scatter(b):
        pltpu.make_async_copy(hbuf[b], acc.at[dsc[b]], ssem[b]).wait()

    def issue_loads(b, off):
        pltpu.async_copy(src_hbm.at[pl.ds(off, CB)], sidx[b], lsem[b])
        pltpu.async_copy(att_hbm.at[pl.ds(off * 16, CB * 16)], atb[b], lsem[b])

    def wait_loads(b):
        pltpu.make_async_copy(src_hbm.at[pl.ds(0, CB)], sidx[b], lsem[b]).wait()
        pltpu.make_async_copy(att_hbm.at[pl.ds(0, CB * 16)], atb[b], lsem[b]).wait()

    def issue_gather(b, fb):
        def addoff(j, _):
            sgi[b][pl.ds(j * 16, 16)] = sidx[b][pl.ds(j * 16, 16)] + fb * N
            return 0
        lax.fori_loop(0, CB // 16, addoff, 0)
        pltpu.async_copy(hsp_hbm.at[sgi[b]], hbuf[b], sem[b])

    def wait_gather(b):
        pltpu.make_async_copy(hsp_hbm.at[sgi[b]], hbuf[b], sem[b]).wait()

    for k in range(HEADS // NSC):
        fb = c * (HEADS // NSC) + k

        def zrow(r, _):
            for v in range(AW // 16):
                hbuf0[r, pl.ds(v * 16, 16)] = jnp.zeros((16,), jnp.float32)
            return 0
        lax.fori_loop(0, CB, zrow, 0)
        for z in range(STRIPE_A // CB):
            pltpu.sync_copy(hbuf0, acc.at[pl.ds(s * STRIPE_A + z * CB, CB)])
        plsc.subcore_barrier()

        base = s * estripe
        pltpu.sync_copy(dst_hbm.at[pl.ds(base, CB)], didx[0])
        issue_loads(0, base)
        wait_loads(0)
        issue_gather(0, fb)
        pltpu.sync_copy(dst_hbm.at[pl.ds(base + CB, CB)], didx[1])
        issue_loads(1, base + CB)

        def pair(q2, _):
            q = q2 * 2

            def half(b, qa):
                qn2 = jnp.minimum(qa + 2, NCH - 1)
                wait_loads(1 - b)
                if b == 0:
                    @pl.when(q2 > 0)
                    def _():
                        wait_scatter(1)
                else:
                    wait_scatter(0)
                issue_gather(1 - b, fb)
                wait_gather(b)

                @pl.when(c == 0)
                def _():
                    _scale_rows(hbuf[b], atb[b], k)

                @pl.when(c == 1)
                def _():
                    _scale_rows(hbuf[b], atb[b], HEADS // NSC + k)

                def snap(j, _):
                    dsc[b][pl.ds(j * 16, 16)] = didx[b][pl.ds(j * 16, 16)]
                    return 0
                lax.fori_loop(0, CB // 16, snap, 0)
                pltpu.async_copy(hbuf[b], acc.at[dsc[b]], ssem[b], add=True)
                issue_loads(b, base + qn2 * CB)
                pltpu.async_copy(dst_hbm.at[pl.ds(base + qn2 * CB, CB)],
                                 didx[b], dsem[b])
                pltpu.make_async_copy(dst_hbm.at[pl.ds(0, CB)], didx[b],
                                      dsem[b]).wait()
            half(0, q)
            half(1, q + 1)
            return 0
        lax.fori_loop(0, NCH // 2, pair, 0)
        wait_loads(1)
        wait_gather(0)
        wait_scatter(1)
        plsc.subcore_barrier()

        pltpu.sync_copy(acc.at[pl.ds(s * STRIPE_A, STRIPE_A)],

                        out_hbm.at[fb, pl.ds(s * STRIPE_A, STRIPE_A)])
        plsc.subcore_barrier()


_agg1_kernel = functools.partial(
    pl.kernel,
    out_type=jax.ShapeDtypeStruct((HEADS, NPAD, AW), jnp.float32),
    mesh=_sc_mesh,
    scratch_types=[
        pltpu.VMEM((CB,), jnp.int32),
        pltpu.VMEM((CB,), jnp.int32),
        pltpu.VMEM((CB,), jnp.int32),
        pltpu.VMEM((CB,), jnp.int32),
        pltpu.VMEM((CB,), jnp.int32),
        pltpu.VMEM((CB,), jnp.int32),
        pltpu.VMEM((CB * 16,), jnp.float32),
        pltpu.VMEM((CB * 16,), jnp.float32),
        pltpu.VMEM((CB, AW), jnp.float32),
        pltpu.VMEM((CB, AW), jnp.float32),
        pltpu.VMEM((CB,), jnp.int32),
        pltpu.VMEM((CB,), jnp.int32),
        pltpu.VMEM_SHARED((NPAD, AW), jnp.float32),
        pltpu.SemaphoreType.DMA,
        pltpu.SemaphoreType.DMA,
        pltpu.SemaphoreType.DMA,
        pltpu.SemaphoreType.DMA,
        pltpu.SemaphoreType.DMA,
        pltpu.SemaphoreType.DMA,
        pltpu.SemaphoreType.DMA,
        pltpu.SemaphoreType.DMA,
    ],
)(_agg1_body)


def _agg2_body(src_hbm, dst_hbm, att_hbm, h_hbm, out_hbm,
               sidx0, sidx1, didx0, didx1, atb0, atb1, hbuf0, hbuf1,
               acc, sem0, sem1):
    c = lax.axis_index("c")
    s = lax.axis_index("s")
    wid = s * NSC + c
    NCH = C_TILE // CB
    sidx = (sidx0, sidx1)
    didx = (didx0, didx1)
    atb = (atb0, atb1)
    hbuf = (hbuf0, hbuf1)
    sem = (sem0, sem1)

    def start_chunk(b, off):
        pltpu.sync_copy(src_hbm.at[pl.ds(off, CB)], sidx[b])
        pltpu.sync_copy(dst_hbm.at[pl.ds(off, CB)], didx[b])
        pltpu.sync_copy(att_hbm.at[pl.ds(off * 16, CB * 16)], atb[b])
        pltpu.async_copy(h_hbm.at[sidx[b]], hbuf[b], sem[b])

    def wait_chunk(b):
        pltpu.make_async_copy(h_hbm.at[sidx[b]], hbuf[b], sem[b]).wait()

    def zrow(r, _):
        for v in range(AW // 16):
            hbuf0[r, pl.ds(v * 16, 16)] = jnp.zeros((16,), jnp.float32)
        return 0
    lax.fori_loop(0, CB, zrow, 0)
    for z in range(STRIPE_A // CB):
        pltpu.sync_copy(hbuf0, acc.at[pl.ds(s * STRIPE_A + z * CB, CB)])
    plsc.subcore_barrier()

    base = wid * C_TILE
    start_chunk(0, base)

    def pair(q2, _):
        q = q2 * 2

        def half(b, qa):
            qn = jnp.minimum(qa + 1, NCH - 1)
            start_chunk(1 - b, base + qn * CB)
            wait_chunk(b)
            _scale_rows(hbuf[b], atb[b], 0)
            pltpu.sync_copy(hbuf[b], acc.at[didx[b]], add=True)
        half(0, q)
        half(1, q + 1)
        return 0
    lax.fori_loop(0, NCH // 2, pair, 0)
    wait_chunk(0)
    plsc.subcore_barrier()

    pltpu.sync_copy(acc.at[pl.ds(s * STRIPE_A, STRIPE_A)],
                    out_hbm.at[c, pl.ds(s * STRIPE_A, STRIPE_A)])


_agg2_kernel = functools.partial(
    pl.kernel,
    out_type=jax.ShapeDtypeStruct((NSC, NPAD, AW), jnp.float32),
    mesh=_sc_mesh,
    scratch_types=(
        [pltpu.VMEM((CB,), jnp.int32)] * 4
        + [pltpu.VMEM((CB * 16,), jnp.float32)] * 2
        + [pltpu.VMEM((CB, AW), jnp.float32)] * 2
        + [pltpu.VMEM_SHARED((NPAD, AW), jnp.float32)]
        + [pltpu.SemaphoreType.DMA] * 2
    ),
)(_agg2_body)


_COMB_BLK = 1024


def _comb_body(al_ref, d0_ref, d1_ref, dd_ref):
    colmask = lax.broadcasted_iota(jnp.int32, (_COMB_BLK, AW), 1) < TW
    dd_ref[...] = jnp.where(colmask, d0_ref[...] + d1_ref[...], al_ref[...])


def _combine(alp, d0, d1):
    return pl.pallas_call(
        _comb_body,
        grid=(NPAD // _COMB_BLK,),
        in_specs=[pl.BlockSpec((_COMB_BLK, AW), lambda i: (i, 0))] * 3,
        out_specs=pl.BlockSpec((_COMB_BLK, AW), lambda i: (i, 0)),
        out_shape=jax.ShapeDtypeStruct((NPAD, AW), jnp.float32),
    )(alp, d0, d1)


def _mm1_body(x_ref, w_ref, am_ref, h_ref, al_ref):
    h = jnp.dot(x_ref[...], w_ref[...], preferred_element_type=jnp.float32)
    h_ref[...] = h
    al_ref[...] = jnp.dot(h, am_ref[...], preferred_element_type=jnp.float32)


def _mm1(x, W, a_mat, d_in, d_out):
    grid = N // ROW_BLK
    return pl.pallas_call(
        _mm1_body,
        grid=(grid,),
        in_specs=[
            pl.BlockSpec((ROW_BLK, d_in), lambda i: (i, 0)),
            pl.BlockSpec((d_in, d_out), lambda i: (0, 0)),
            pl.BlockSpec((d_out, AW), lambda i: (0, 0)),
        ],
        out_specs=[
            pl.BlockSpec((ROW_BLK, d_out), lambda i: (i, 0)),
            pl.BlockSpec((ROW_BLK, AW), lambda i: (i, 0)),
        ],
        out_shape=[
            jax.ShapeDtypeStruct((N, d_out), jnp.float32),
            jax.ShapeDtypeStruct((N, AW), jnp.float32),
        ],
    )(x, W, a_mat)


def _mid_body(agg_ref, b1_ref, w2_ref, am_ref, g_ref, al_ref):
    h1 = jnp.maximum(agg_ref[...] + b1_ref[...], 0.0)
    g = jnp.dot(h1, w2_ref[...], preferred_element_type=jnp.float32)
    g_ref[...] = g
    al_ref[...] = jnp.dot(g, am_ref[...], preferred_element_type=jnp.float32)


def _mid(agg1, b1, W2, a_mat2):
    grid = N // ROW_BLK
    return pl.pallas_call(
        _mid_body,
        grid=(grid,),
        in_specs=[
            pl.BlockSpec((ROW_BLK, HEADS * D), lambda i: (i, 0)),
            pl.BlockSpec((1, HEADS * D), lambda i: (0, 0)),
            pl.BlockSpec((HEADS * D, OUT), lambda i: (0, 0)),
            pl.BlockSpec((OUT, AW), lambda i: (0, 0)),
        ],
        out_specs=[
            pl.BlockSpec((ROW_BLK, OUT), lambda i: (i, 0)),
            pl.BlockSpec((ROW_BLK, AW), lambda i: (i, 0)),
        ],
        out_shape=[
            jax.ShapeDtypeStruct((N, OUT), jnp.float32),
            jax.ShapeDtypeStruct((N, AW), jnp.float32),
        ],
    )(agg1, b1.reshape(1, HEADS * D), W2, a_mat2)


def _fin_body(p0_ref, p1_ref, b2_ref, fcw_ref, fcb_ref, y_ref, gr_ref, mx_ref):
    i = pl.program_id(0)
    h2 = jnp.maximum(p0_ref[...] + p1_ref[...] + b2_ref[...], 0.0)
    y_ref[...] = (
        jnp.dot(h2, fcw_ref[...], preferred_element_type=jnp.float32)
        + fcb_ref[...]
    )
    blockmax = jnp.max(h2, axis=0, keepdims=True)

    @pl.when(i == 0)
    def _():
        mx_ref[...] = blockmax

    @pl.when(i > 0)
    def _():
        mx_ref[...] = jnp.maximum(mx_ref[...], blockmax)

    @pl.when(i == pl.num_programs(0) - 1)
    def _():
        gr_ref[...] = (
            jnp.dot(mx_ref[...], fcw_ref[...], preferred_element_type=jnp.float32)
            + fcb_ref[...]
        )


def _fin(p0, p1, b2, fcW, fcb):
    grid = N // ROW_BLK
    return pl.pallas_call(
        _fin_body,
        grid=(grid,),
        in_specs=[
            pl.BlockSpec((ROW_BLK, OUT), lambda i: (i, 0)),
            pl.BlockSpec((ROW_BLK, OUT), lambda i: (i, 0)),
            pl.BlockSpec((1, OUT), lambda i: (0, 0)),
            pl.BlockSpec((OUT, OUT), lambda i: (0, 0)),
            pl.BlockSpec((1, OUT), lambda i: (0, 0)),
        ],
        out_specs=[
            pl.BlockSpec((ROW_BLK, OUT), lambda i: (i, 0)),
            pl.BlockSpec((1, OUT), lambda i: (0, 0)),
        ],
        out_shape=[
            jax.ShapeDtypeStruct((N, OUT), jnp.float32),
            jax.ShapeDtypeStruct((1, OUT), jnp.float32),
        ],
        scratch_shapes=[pltpu.VMEM((1, OUT), jnp.float32)],
    )(p0, p1, b2.reshape(1, OUT), fcW, fcb.reshape(1, OUT))


def _pack_alpha(a_src, a_dst, d_out, heads, head_dim):
    m = jnp.zeros((d_out, AW), jnp.float32)
    for h in range(heads):
        m = m.at[h * head_dim:(h + 1) * head_dim, h].set(a_src[h])
        m = m.at[h * head_dim:(h + 1) * head_dim, TW + h].set(a_dst[h])
    return m


def kernel(x, edge_index, graph_id, W1, a_src1, a_dst1, b1, W2, a_src2,
           a_dst2, b2, fcW, fcb):
    loops = jnp.arange(N, dtype=edge_index.dtype)
    pad = jnp.zeros((EP - E - N,), jnp.int32)
    src = jnp.concatenate([edge_index[0], loops])
    dst = jnp.concatenate([edge_index[1], loops])
    src_p = jnp.concatenate([src, pad])
    dst_p = jnp.concatenate([dst, pad + N])

    a_mat1 = _pack_alpha(a_src1, a_dst1, HEADS * D, HEADS, D)
    a_mat2 = _pack_alpha(a_src2, a_dst2, OUT, 1, OUT)

    def tables(al):
        return jnp.pad(al, ((0, NPAD - N), (0, 0)))

    h1mm, al1 = _mm1(x, W1, a_mat1, D, HEADS * D)
    al1p = tables(al1)
    dns1 = _denom_kernel(src_p, dst_p, al1p)
    dd1 = _combine(al1p, dns1[0], dns1[1])
    att1 = _att_kernel(src_p, dst_p, al1p, dd1)
    hsp = h1mm.reshape(N, HEADS, D).swapaxes(0, 1).reshape(HEADS * N, D)
    out1 = _agg1_kernel(src_p, dst_p, att1, hsp)
    agg1 = out1[:, :N, :].swapaxes(0, 1).reshape(N, HEADS * D)
    g2, al2 = _mid(agg1, b1, W2, a_mat2)
    al2p = tables(al2)
    dns2 = _denom_kernel(src_p, dst_p, al2p)
    dd2 = _combine(al2p, dns2[0], dns2[1])
    att2 = _att_kernel(src_p, dst_p, al2p, dd2)
    agg2p = _agg2_kernel(src_p, dst_p, att2, g2)
    y, gr = _fin(agg2p[0, :N], agg2p[1, :N], b2, fcW, fcb)
    return y[None, :, :], gr
